# Initial kernel scaffold; baseline (speedup 1.0000x reference)
import functools

import jax
import jax.numpy as jnp
from jax import lax
from jax.experimental import pallas as pl
from jax.experimental.pallas import tpu as pltpu
from jax.experimental.pallas import tpu_sc as plsc

N = 10000
NP = 10240
E = 320000
D = 128
F = 16
NTILE = 16
CHUNK = 128
NCHUNK = 157
EPT = NCHUNK * CHUNK
EP = EPT * NTILE
NPT = NP // NTILE
G2 = 2 * NP

_mesh = plsc.VectorSubcoreMesh(core_axis_name="core", subcore_axis_name="subcore")


@functools.partial(
    pl.kernel,
    out_type=jax.ShapeDtypeStruct((G2,), jnp.float32),
    mesh=_mesh,
    scratch_types=[
        pltpu.VMEM((NCHUNK, CHUNK), jnp.int32),
        pltpu.VMEM((CHUNK,), jnp.float32),
        pltpu.VMEM((NPT,), jnp.float32),
        pltpu.VMEM_SHARED((G2,), jnp.float32),
    ],
)
def _sc_degree(dst_hbm, deg_hbm, dst_v, ones_v, zero_v, deg_sh):
    c = lax.axis_index("core")
    s = lax.axis_index("subcore")
    base = c * NP + s * NPT

    @pl.loop(0, CHUNK // 16)
    def _(i):
        ones_v[pl.ds(i * 16, 16)] = jnp.full((16,), 1.0, jnp.float32)

    @pl.loop(0, NPT // 16)
    def _(i):
        zero_v[pl.ds(i * 16, 16)] = jnp.zeros((16,), jnp.float32)

    pltpu.sync_copy(zero_v, deg_sh.at[pl.ds(base, NPT)])
    pltpu.sync_copy(dst_hbm.at[c, s], dst_v)
    plsc.subcore_barrier()

    @pl.loop(0, NCHUNK)
    def _(j):
        pltpu.sync_copy(ones_v, deg_sh.at[dst_v.at[j]], add=True)

    plsc.subcore_barrier()
    pltpu.sync_copy(deg_sh.at[pl.ds(base, NPT)], deg_hbm.at[pl.ds(base, NPT)])


@functools.partial(
    pl.kernel,
    out_type=(
        jax.ShapeDtypeStruct((G2,), jnp.float32),
        jax.ShapeDtypeStruct((G2, F), jnp.float32),
    ),
    mesh=_mesh,
    scratch_types=[
        pltpu.VMEM((NCHUNK, CHUNK), jnp.int32),
        pltpu.VMEM((NCHUNK, CHUNK), jnp.int32),
        pltpu.VMEM((NCHUNK, CHUNK), jnp.float32),
        pltpu.VMEM((G2,), jnp.float32),
        pltpu.VMEM((CHUNK, F), jnp.float32),
        pltpu.VMEM((NPT, F), jnp.float32),
        pltpu.VMEM((NPT,), jnp.float32),
        pltpu.VMEM_SHARED((G2,), jnp.float32),
        pltpu.VMEM_SHARED((G2, F), jnp.float32),
    ],
)
def _sc_agg(src_hbm, dst_hbm, dinv_hbm, g_hbm, t_hbm, acc_hbm,
            src_v, dst_v, tval_v, dinv_v, rows_v, zrows_v, zero_v,
            t_sh, acc_sh):
    c = lax.axis_index("core")
    s = lax.axis_index("subcore")
    base = c * NP + s * NPT

    @pl.loop(0, NPT)
    def _(i):
        zrows_v[i, :] = jnp.zeros((F,), jnp.float32)

    @pl.loop(0, NPT // 16)
    def _(i):
        zero_v[pl.ds(i * 16, 16)] = jnp.zeros((16,), jnp.float32)

    pltpu.sync_copy(zrows_v, acc_sh.at[pl.ds(base, NPT)])
    pltpu.sync_copy(zero_v, t_sh.at[pl.ds(base, NPT)])
    pltpu.sync_copy(src_hbm.at[c, s], src_v)
    pltpu.sync_copy(dst_hbm.at[c, s], dst_v)
    pltpu.sync_copy(dinv_hbm, dinv_v)

    @pl.loop(0, NCHUNK)
    def _(j):
        @pl.loop(0, CHUNK // 16)
        def _(k):
            idx = dst_v[j, pl.ds(k * 16, 16)]
            tval_v[j, pl.ds(k * 16, 16)] = plsc.load_gather(dinv_v, [idx])

    plsc.subcore_barrier()

    @pl.loop(0, NCHUNK)
    def _(j):
        pltpu.sync_copy(g_hbm.at[src_v.at[j]], rows_v)
        pltpu.sync_copy(rows_v, acc_sh.at[dst_v.at[j]], add=True)
        pltpu.sync_copy(tval_v.at[j], t_sh.at[src_v.at[j]], add=True)

    plsc.subcore_barrier()
    pltpu.sync_copy(t_sh.at[pl.ds(base, NPT)], t_hbm.at[pl.ds(base, NPT)])
    pltpu.sync_copy(acc_sh.at[pl.ds(base, NPT)], acc_hbm.at[pl.ds(base, NPT)])


def _mm_body(x_ref, w_ref, o_ref):
    o_ref[0] = jnp.dot(x_ref[0], w_ref[0], preferred_element_type=jnp.float32)


def _scale_body(deg_ref, h_ref, dinv_ref, invdeg_ref, g_ref):
    deg = deg_ref[...] + 1.0
    dinv = lax.rsqrt(deg)
    dinv_ref[...] = dinv
    invdeg_ref[...] = 1.0 / deg
    g_ref[...] = h_ref[...] * dinv[..., None]


def _head_body(h_ref, acc_ref, t_ref, dinv_ref, invdeg_ref,
               bpin_ref, blin_ref, wpout_ref, bpout_ref, wlout_ref, blout_ref,
               w1_ref, b1_ref, w2_ref, b2_ref, w3_ref, b3_ref, act_ref, o_ref):
    mask = (lax.broadcasted_iota(jnp.int32, (NP, 1), 0) < N).astype(jnp.float32)

    def pool(gi, b_vec, w_out, b_out):
        dinv = dinv_ref[gi][:, None]
        invdeg = invdeg_ref[gi][:, None]
        out1 = dinv * acc_ref[gi] + invdeg * h_ref[gi] + b_vec
        h1 = jnp.maximum(out1, 0.0)
        cvec = (dinv * t_ref[gi][:, None] + invdeg) * mask
        s_vec = jnp.sum(cvec * h1, axis=0, keepdims=True)
        return jnp.dot(s_vec / float(N), w_out,
                       preferred_element_type=jnp.float32) + b_out

    p = pool(0, bpin_ref[...], wpout_ref[...], bpout_ref[...])
    l = pool(1, blin_ref[...], wlout_ref[...], blout_ref[...])
    fp = jnp.maximum(
        jnp.dot(p, w1_ref[0:50, :], preferred_element_type=jnp.float32)
        + jnp.dot(l, w1_ref[50:100, :], preferred_element_type=jnp.float32)
        + b1_ref[...], 0.0)
    pol = (jnp.dot(fp, w2_ref[0:60, :], preferred_element_type=jnp.float32)
           + jnp.dot(act_ref[...], w2_ref[60:100, :],
                     preferred_element_type=jnp.float32)
           + b2_ref[...])
    o_ref[...] = jnp.dot(jnp.maximum(pol, 0.0), w3_ref[...],
                         preferred_element_type=jnp.float32) + b3_ref[...]


def _prep_edges(ei, graph):
    pad = jnp.full((EP - E,), N, jnp.int32)
    off = graph * NP
    src = jnp.concatenate([ei[0], pad]) + off
    dst = jnp.concatenate([ei[1], pad]) + off
    return (src.reshape(NTILE, NCHUNK, CHUNK),
            dst.reshape(NTILE, NCHUNK, CHUNK))


def kernel(protein_x, protein_edge_index, ligand_x, ligand_edge_index, action,
           W_pin, b_pin, W_pout, b_pout, W_lin, b_lin, W_lout, b_lout,
           W1, b1, W2, b2, W3, b3):
    f32 = jnp.float32
    src_p, dst_p = _prep_edges(protein_edge_index, 0)
    src_l, dst_l = _prep_edges(ligand_edge_index, 1)
    srcs = jnp.stack([src_p, src_l])
    dsts = jnp.stack([dst_p, dst_l])
    xs = jnp.stack([
        jnp.pad(protein_x, ((0, NP - N), (0, 0))),
        jnp.pad(ligand_x, ((0, NP - N), (0, 0))),
    ])
    ws = jnp.stack([W_pin, W_lin])

    hs = pl.pallas_call(
        _mm_body,
        grid=(2, NP // 1024),
        in_specs=[
            pl.BlockSpec((1, 1024, D), lambda g, i: (g, i, 0)),
            pl.BlockSpec((1, D, F), lambda g, i: (g, 0, 0)),
        ],
        out_specs=pl.BlockSpec((1, 1024, F), lambda g, i: (g, i, 0)),
        out_shape=jax.ShapeDtypeStruct((2, NP, F), f32),
    )(xs, ws)

    deg = _sc_degree(dsts)

    dinv, invdeg, g = pl.pallas_call(
        _scale_body,
        out_shape=[jax.ShapeDtypeStruct((2, NP), f32),
                   jax.ShapeDtypeStruct((2, NP), f32),
                   jax.ShapeDtypeStruct((2, NP, F), f32)],
    )(deg.reshape(2, NP), hs)

    t, acc = _sc_agg(srcs, dsts, dinv.reshape(G2), g.reshape(G2, F))

    out = pl.pallas_call(
        _head_body,
        out_shape=jax.ShapeDtypeStruct((1, 1), f32),
    )(hs, acc.reshape(2, NP, F), t.reshape(2, NP), dinv, invdeg,
      b_pin.reshape(1, F), b_lin.reshape(1, F), W_pout, b_pout.reshape(1, 50),
      W_lout, b_lout.reshape(1, 50), W1, b1.reshape(1, 60), W2,
      b2.reshape(1, 10), W3, b3.reshape(1, 1), action)
    return out


# speedup vs baseline: 49.1998x; 49.1998x over previous
import dataclasses
import functools

import jax
import jax.numpy as jnp
from jax import lax
from jax.experimental import pallas as pl
from jax.experimental.pallas import tpu as pltpu
from jax.experimental.pallas import tpu_sc as plsc

N = 10000
NP = 10240
E = 320000
D = 128
F = 16
NTILE = 16
CHUNK = 128
NCHUNK = 157
EPT = NCHUNK * CHUNK
EP = EPT * NTILE
NPT = NP // NTILE
G2 = 2 * NP

_mesh = plsc.VectorSubcoreMesh(core_axis_name="core", subcore_axis_name="subcore")

_sc_params = pltpu.CompilerParams(
    needs_layout_passes=False, use_tc_tiling_on_sc=False)


@functools.partial(
    pl.kernel,
    out_type=jax.ShapeDtypeStruct((G2,), jnp.float32),
    mesh=_mesh,
    scratch_types=[
        pltpu.VMEM((NCHUNK, CHUNK), jnp.int32),
        pltpu.VMEM((CHUNK,), jnp.float32),
        pltpu.VMEM((NPT,), jnp.float32),
        pltpu.VMEM_SHARED((G2,), jnp.float32),
    ],
)
def _sc_degree(dst_hbm, deg_hbm, dst_v, ones_v, zero_v, deg_sh):
    c = lax.axis_index("core")
    s = lax.axis_index("subcore")
    base = c * NP + s * NPT

    @pl.loop(0, CHUNK // 16)
    def _(i):
        ones_v[pl.ds(i * 16, 16)] = jnp.full((16,), 1.0, jnp.float32)

    @pl.loop(0, NPT // 16)
    def _(i):
        zero_v[pl.ds(i * 16, 16)] = jnp.zeros((16,), jnp.float32)

    pltpu.sync_copy(zero_v, deg_sh.at[pl.ds(base, NPT)])
    pltpu.sync_copy(dst_hbm.at[c, s], dst_v)
    plsc.subcore_barrier()

    @pl.loop(0, NCHUNK)
    def _(j):
        pltpu.sync_copy(ones_v, deg_sh.at[dst_v.at[j]], add=True)

    plsc.subcore_barrier()
    pltpu.sync_copy(deg_sh.at[pl.ds(base, NPT)], deg_hbm.at[pl.ds(base, NPT)])


@functools.partial(
    pl.kernel,
    out_type=(
        jax.ShapeDtypeStruct((G2,), jnp.float32),
        jax.ShapeDtypeStruct((G2, F), jnp.float32),
    ),
    mesh=_mesh,
    scratch_types=[
        pltpu.VMEM((NCHUNK, CHUNK), jnp.int32),
        pltpu.VMEM((NCHUNK, CHUNK), jnp.int32),
        pltpu.VMEM((NCHUNK, CHUNK), jnp.float32),
        pltpu.VMEM((G2,), jnp.float32),
        pltpu.VMEM((CHUNK, F), jnp.float32),
        pltpu.VMEM((NPT, F), jnp.float32),
        pltpu.VMEM((NPT,), jnp.float32),
        pltpu.VMEM_SHARED((G2,), jnp.float32),
        pltpu.VMEM_SHARED((G2, F), jnp.float32),
    ],
    compiler_params=_sc_params,
)
def _sc_agg(src_hbm, dst_hbm, dinv_hbm, g_hbm, t_hbm, acc_hbm,
            src_v, dst_v, tval_v, dinv_v, rows_v, zrows_v, zero_v,
            t_sh, acc_sh):
    c = lax.axis_index("core")
    s = lax.axis_index("subcore")
    base = c * NP + s * NPT

    @pl.loop(0, NPT)
    def _(i):
        zrows_v[i, :] = jnp.zeros((F,), jnp.float32)

    @pl.loop(0, NPT // 16)
    def _(i):
        zero_v[pl.ds(i * 16, 16)] = jnp.zeros((16,), jnp.float32)

    pltpu.sync_copy(zrows_v, acc_sh.at[pl.ds(base, NPT)])
    pltpu.sync_copy(zero_v, t_sh.at[pl.ds(base, NPT)])
    pltpu.sync_copy(src_hbm.at[c, s], src_v)
    pltpu.sync_copy(dst_hbm.at[c, s], dst_v)
    pltpu.sync_copy(dinv_hbm, dinv_v)

    @pl.loop(0, NCHUNK)
    def _(j):
        @pl.loop(0, CHUNK // 16)
        def _(k):
            idx = dst_v[j, pl.ds(k * 16, 16)]
            tval_v[j, pl.ds(k * 16, 16)] = plsc.load_gather(dinv_v, [idx])

    plsc.subcore_barrier()

    @pl.loop(0, NCHUNK)
    def _(j):
        pltpu.sync_copy(g_hbm.at[src_v.at[j]], rows_v)
        pltpu.sync_copy(rows_v, acc_sh.at[dst_v.at[j]], add=True)
        pltpu.sync_copy(tval_v.at[j], t_sh.at[src_v.at[j]], add=True)

    plsc.subcore_barrier()
    pltpu.sync_copy(t_sh.at[pl.ds(base, NPT)], t_hbm.at[pl.ds(base, NPT)])
    pltpu.sync_copy(acc_sh.at[pl.ds(base, NPT)], acc_hbm.at[pl.ds(base, NPT)])


def _mm_body(x_ref, w_ref, o_ref):
    o_ref[0] = jnp.dot(x_ref[0], w_ref[0], preferred_element_type=jnp.float32)


def _scale_body(deg_ref, h_ref, dinv_ref, invdeg_ref, g_ref):
    deg = deg_ref[...] + 1.0
    dinv = lax.rsqrt(deg)
    dinv_ref[...] = dinv
    invdeg_ref[...] = 1.0 / deg
    g_ref[...] = h_ref[...] * dinv[..., None]


def _head_body(h_ref, acc_ref, t_ref, dinv_ref, invdeg_ref,
               bpin_ref, blin_ref, wpout_ref, bpout_ref, wlout_ref, blout_ref,
               w1_ref, b1_ref, w2_ref, b2_ref, w3_ref, b3_ref, act_ref, o_ref):
    mask = (lax.broadcasted_iota(jnp.int32, (NP, 1), 0) < N).astype(jnp.float32)

    def pool(gi, b_vec, w_out, b_out):
        dinv = dinv_ref[gi][:, None]
        invdeg = invdeg_ref[gi][:, None]
        out1 = dinv * acc_ref[gi] + invdeg * h_ref[gi] + b_vec
        h1 = jnp.maximum(out1, 0.0)
        cvec = (dinv * t_ref[gi][:, None] + invdeg) * mask
        s_vec = jnp.sum(cvec * h1, axis=0, keepdims=True)
        return jnp.dot(s_vec / float(N), w_out,
                       preferred_element_type=jnp.float32) + b_out

    p = pool(0, bpin_ref[...], wpout_ref[...], bpout_ref[...])
    l = pool(1, blin_ref[...], wlout_ref[...], blout_ref[...])
    fp = jnp.maximum(
        jnp.dot(p, w1_ref[0:50, :], preferred_element_type=jnp.float32)
        + jnp.dot(l, w1_ref[50:100, :], preferred_element_type=jnp.float32)
        + b1_ref[...], 0.0)
    pol = (jnp.dot(fp, w2_ref[0:60, :], preferred_element_type=jnp.float32)
           + jnp.dot(act_ref[...], w2_ref[60:100, :],
                     preferred_element_type=jnp.float32)
           + b2_ref[...])
    o_ref[...] = jnp.dot(jnp.maximum(pol, 0.0), w3_ref[...],
                         preferred_element_type=jnp.float32) + b3_ref[...]


def _prep_edges(ei, graph):
    pad = jnp.full((EP - E,), N, jnp.int32)
    off = graph * NP
    src = jnp.concatenate([ei[0], pad]) + off
    dst = jnp.concatenate([ei[1], pad]) + off
    return (src.reshape(NTILE, NCHUNK, CHUNK),
            dst.reshape(NTILE, NCHUNK, CHUNK))


def kernel(protein_x, protein_edge_index, ligand_x, ligand_edge_index, action,
           W_pin, b_pin, W_pout, b_pout, W_lin, b_lin, W_lout, b_lout,
           W1, b1, W2, b2, W3, b3):
    f32 = jnp.float32
    src_p, dst_p = _prep_edges(protein_edge_index, 0)
    src_l, dst_l = _prep_edges(ligand_edge_index, 1)
    srcs = jnp.stack([src_p, src_l])
    dsts = jnp.stack([dst_p, dst_l])
    xs = jnp.stack([
        jnp.pad(protein_x, ((0, NP - N), (0, 0))),
        jnp.pad(ligand_x, ((0, NP - N), (0, 0))),
    ])
    ws = jnp.stack([W_pin, W_lin])

    hs = pl.pallas_call(
        _mm_body,
        grid=(2, NP // 1024),
        in_specs=[
            pl.BlockSpec((1, 1024, D), lambda g, i: (g, i, 0)),
            pl.BlockSpec((1, D, F), lambda g, i: (g, 0, 0)),
        ],
        out_specs=pl.BlockSpec((1, 1024, F), lambda g, i: (g, i, 0)),
        out_shape=jax.ShapeDtypeStruct((2, NP, F), f32),
    )(xs, ws)

    deg = _sc_degree(dsts)

    dinv, invdeg, g = pl.pallas_call(
        _scale_body,
        out_shape=[jax.ShapeDtypeStruct((2, NP), f32),
                   jax.ShapeDtypeStruct((2, NP), f32),
                   jax.ShapeDtypeStruct((2, NP, F), f32)],
    )(deg.reshape(2, NP), hs)

    t, acc = _sc_agg(srcs, dsts, dinv.reshape(G2), g.reshape(G2, F))

    out = pl.pallas_call(
        _head_body,
        out_shape=jax.ShapeDtypeStruct((1, 1), f32),
    )(hs, acc.reshape(2, NP, F), t.reshape(2, NP), dinv, invdeg,
      b_pin.reshape(1, F), b_lin.reshape(1, F), W_pout, b_pout.reshape(1, 50),
      W_lout, b_lout.reshape(1, 50), W1, b1.reshape(1, 60), W2,
      b2.reshape(1, 10), W3, b3.reshape(1, 1), action)
    return out


# speedup vs baseline: 53.5291x; 1.0880x over previous
import dataclasses
import functools

import jax
import jax.numpy as jnp
from jax import lax
from jax.experimental import pallas as pl
from jax.experimental.pallas import tpu as pltpu
from jax.experimental.pallas import tpu_sc as plsc

N = 10000
NP = 10240
E = 320000
D = 128
F = 16
NTILE = 16
CHUNK = 128
NCHUNK = 158
EPT = NCHUNK * CHUNK
EP = EPT * NTILE
NPT = NP // NTILE
G2 = 2 * NP

_mesh = plsc.VectorSubcoreMesh(core_axis_name="core", subcore_axis_name="subcore")

_sc_params = pltpu.CompilerParams(
    needs_layout_passes=False, use_tc_tiling_on_sc=False)


@functools.partial(
    pl.kernel,
    out_type=jax.ShapeDtypeStruct((G2,), jnp.float32),
    mesh=_mesh,
    scratch_types=[
        pltpu.VMEM((NCHUNK, CHUNK), jnp.int32),
        pltpu.VMEM((CHUNK,), jnp.float32),
        pltpu.VMEM((NPT,), jnp.float32),
        pltpu.VMEM_SHARED((G2,), jnp.float32),
    ],
)
def _sc_degree(dst_hbm, deg_hbm, dst_v, ones_v, zero_v, deg_sh):
    c = lax.axis_index("core")
    s = lax.axis_index("subcore")
    base = c * NP + s * NPT

    @pl.loop(0, CHUNK // 16)
    def _(i):
        ones_v[pl.ds(i * 16, 16)] = jnp.full((16,), 1.0, jnp.float32)

    @pl.loop(0, NPT // 16)
    def _(i):
        zero_v[pl.ds(i * 16, 16)] = jnp.zeros((16,), jnp.float32)

    pltpu.sync_copy(zero_v, deg_sh.at[pl.ds(base, NPT)])
    pltpu.sync_copy(dst_hbm.at[c, s], dst_v)
    plsc.subcore_barrier()

    @pl.loop(0, NCHUNK)
    def _(j):
        pltpu.sync_copy(ones_v, deg_sh.at[dst_v.at[j]], add=True)

    plsc.subcore_barrier()
    pltpu.sync_copy(deg_sh.at[pl.ds(base, NPT)], deg_hbm.at[pl.ds(base, NPT)])


@functools.partial(
    pl.kernel,
    out_type=(
        jax.ShapeDtypeStruct((G2,), jnp.float32),
        jax.ShapeDtypeStruct((G2, F), jnp.float32),
    ),
    mesh=_mesh,
    scratch_types=[
        pltpu.VMEM((NCHUNK, CHUNK), jnp.int32),
        pltpu.VMEM((NCHUNK, CHUNK), jnp.int32),
        pltpu.VMEM((NCHUNK, CHUNK), jnp.float32),
        pltpu.VMEM((G2,), jnp.float32),
        pltpu.VMEM((CHUNK, F), jnp.float32),
        pltpu.VMEM((CHUNK, F), jnp.float32),
        pltpu.VMEM((NPT, F), jnp.float32),
        pltpu.VMEM((NPT,), jnp.float32),
        pltpu.VMEM_SHARED((G2,), jnp.float32),
        pltpu.VMEM_SHARED((G2, F), jnp.float32),
        pltpu.SemaphoreType.DMA,
        pltpu.SemaphoreType.DMA,
    ],
    compiler_params=_sc_params,
)
def _sc_agg(src_hbm, dst_hbm, dinv_hbm, g_hbm, t_hbm, acc_hbm,
            src_v, dst_v, tval_v, dinv_v, rows0_v, rows1_v, zrows_v, zero_v,
            t_sh, acc_sh, gsem, ssem):
    c = lax.axis_index("core")
    s = lax.axis_index("subcore")
    base = c * NP + s * NPT

    @pl.loop(0, NPT)
    def _(i):
        zrows_v[i, :] = jnp.zeros((F,), jnp.float32)

    @pl.loop(0, NPT // 16)
    def _(i):
        zero_v[pl.ds(i * 16, 16)] = jnp.zeros((16,), jnp.float32)

    pltpu.sync_copy(zrows_v, acc_sh.at[pl.ds(base, NPT)])
    pltpu.sync_copy(zero_v, t_sh.at[pl.ds(base, NPT)])
    pltpu.sync_copy(src_hbm.at[c, s], src_v)
    pltpu.sync_copy(dst_hbm.at[c, s], dst_v)
    pltpu.sync_copy(dinv_hbm, dinv_v)

    @pl.loop(0, NCHUNK)
    def _(j):
        @pl.loop(0, CHUNK // 16)
        def _(k):
            idx = dst_v[j, pl.ds(k * 16, 16)]
            tval_v[j, pl.ds(k * 16, 16)] = plsc.load_gather(dinv_v, [idx])

    plsc.subcore_barrier()

    def gather_start(jj, buf):
        pltpu.async_copy(g_hbm.at[src_v.at[jj]], buf, gsem)

    def gather_wait(buf):
        pltpu.make_async_copy(g_hbm.at[src_v.at[0]], buf, gsem).wait()

    def process(jj, buf):
        desc = pltpu.async_copy(buf, acc_sh.at[dst_v.at[jj]], ssem, add=True)
        pltpu.sync_copy(tval_v.at[jj], t_sh.at[src_v.at[jj]], add=True)
        desc.wait()

    gather_start(0, rows0_v)

    @pl.loop(0, NCHUNK, step=2)
    def _(j):
        gather_wait(rows0_v)
        gather_start(j + 1, rows1_v)
        process(j, rows0_v)
        gather_wait(rows1_v)

        @pl.when(j + 2 < NCHUNK)
        def _():
            gather_start(j + 2, rows0_v)

        process(j + 1, rows1_v)

    plsc.subcore_barrier()
    pltpu.sync_copy(t_sh.at[pl.ds(base, NPT)], t_hbm.at[pl.ds(base, NPT)])
    pltpu.sync_copy(acc_sh.at[pl.ds(base, NPT)], acc_hbm.at[pl.ds(base, NPT)])


def _mm_body(x_ref, w_ref, o_ref):
    o_ref[0] = jnp.dot(x_ref[0], w_ref[0], preferred_element_type=jnp.float32)


def _scale_body(deg_ref, h_ref, dinv_ref, invdeg_ref, g_ref):
    deg = deg_ref[...] + 1.0
    dinv = lax.rsqrt(deg)
    dinv_ref[...] = dinv
    invdeg_ref[...] = 1.0 / deg
    g_ref[...] = h_ref[...] * dinv[..., None]


def _head_body(h_ref, acc_ref, t_ref, dinv_ref, invdeg_ref,
               bpin_ref, blin_ref, wpout_ref, bpout_ref, wlout_ref, blout_ref,
               w1_ref, b1_ref, w2_ref, b2_ref, w3_ref, b3_ref, act_ref, o_ref):
    mask = (lax.broadcasted_iota(jnp.int32, (NP, 1), 0) < N).astype(jnp.float32)

    def pool(gi, b_vec, w_out, b_out):
        dinv = dinv_ref[gi][:, None]
        invdeg = invdeg_ref[gi][:, None]
        out1 = dinv * acc_ref[gi] + invdeg * h_ref[gi] + b_vec
        h1 = jnp.maximum(out1, 0.0)
        cvec = (dinv * t_ref[gi][:, None] + invdeg) * mask
        s_vec = jnp.sum(cvec * h1, axis=0, keepdims=True)
        return jnp.dot(s_vec / float(N), w_out,
                       preferred_element_type=jnp.float32) + b_out

    p = pool(0, bpin_ref[...], wpout_ref[...], bpout_ref[...])
    l = pool(1, blin_ref[...], wlout_ref[...], blout_ref[...])
    fp = jnp.maximum(
        jnp.dot(p, w1_ref[0:50, :], preferred_element_type=jnp.float32)
        + jnp.dot(l, w1_ref[50:100, :], preferred_element_type=jnp.float32)
        + b1_ref[...], 0.0)
    pol = (jnp.dot(fp, w2_ref[0:60, :], preferred_element_type=jnp.float32)
           + jnp.dot(act_ref[...], w2_ref[60:100, :],
                     preferred_element_type=jnp.float32)
           + b2_ref[...])
    o_ref[...] = jnp.dot(jnp.maximum(pol, 0.0), w3_ref[...],
                         preferred_element_type=jnp.float32) + b3_ref[...]


def _prep_edges(ei, graph):
    pad = jnp.full((EP - E,), N, jnp.int32)
    off = graph * NP
    src = jnp.concatenate([ei[0], pad]) + off
    dst = jnp.concatenate([ei[1], pad]) + off
    return (src.reshape(NTILE, NCHUNK, CHUNK),
            dst.reshape(NTILE, NCHUNK, CHUNK))


def kernel(protein_x, protein_edge_index, ligand_x, ligand_edge_index, action,
           W_pin, b_pin, W_pout, b_pout, W_lin, b_lin, W_lout, b_lout,
           W1, b1, W2, b2, W3, b3):
    f32 = jnp.float32
    src_p, dst_p = _prep_edges(protein_edge_index, 0)
    src_l, dst_l = _prep_edges(ligand_edge_index, 1)
    srcs = jnp.stack([src_p, src_l])
    dsts = jnp.stack([dst_p, dst_l])
    xs = jnp.stack([
        jnp.pad(protein_x, ((0, NP - N), (0, 0))),
        jnp.pad(ligand_x, ((0, NP - N), (0, 0))),
    ])
    ws = jnp.stack([W_pin, W_lin])

    hs = pl.pallas_call(
        _mm_body,
        grid=(2, NP // 1024),
        in_specs=[
            pl.BlockSpec((1, 1024, D), lambda g, i: (g, i, 0)),
            pl.BlockSpec((1, D, F), lambda g, i: (g, 0, 0)),
        ],
        out_specs=pl.BlockSpec((1, 1024, F), lambda g, i: (g, i, 0)),
        out_shape=jax.ShapeDtypeStruct((2, NP, F), f32),
    )(xs, ws)

    deg = _sc_degree(dsts)

    dinv, invdeg, g = pl.pallas_call(
        _scale_body,
        out_shape=[jax.ShapeDtypeStruct((2, NP), f32),
                   jax.ShapeDtypeStruct((2, NP), f32),
                   jax.ShapeDtypeStruct((2, NP, F), f32)],
    )(deg.reshape(2, NP), hs)

    t, acc = _sc_agg(srcs, dsts, dinv.reshape(G2), g.reshape(G2, F))

    out = pl.pallas_call(
        _head_body,
        out_shape=jax.ShapeDtypeStruct((1, 1), f32),
    )(hs, acc.reshape(2, NP, F), t.reshape(2, NP), dinv, invdeg,
      b_pin.reshape(1, F), b_lin.reshape(1, F), W_pout, b_pout.reshape(1, 50),
      W_lout, b_lout.reshape(1, 50), W1, b1.reshape(1, 60), W2,
      b2.reshape(1, 10), W3, b3.reshape(1, 1), action)
    return out


# speedup vs baseline: 61.2789x; 1.1448x over previous
import dataclasses
import functools

import jax
import jax.numpy as jnp
from jax import lax
from jax.experimental import pallas as pl
from jax.experimental.pallas import tpu as pltpu
from jax.experimental.pallas import tpu_sc as plsc

N = 10000
NP = 10240
E = 320000
D = 128
F = 16
NTILE = 16
CHUNK = 512
NCHUNK = 40
EPT = NCHUNK * CHUNK
EP = EPT * NTILE
NPT = NP // NTILE
G2 = 2 * NP

_mesh = plsc.VectorSubcoreMesh(core_axis_name="core", subcore_axis_name="subcore")

_sc_params = pltpu.CompilerParams(
    needs_layout_passes=False, use_tc_tiling_on_sc=False)


@functools.partial(
    pl.kernel,
    out_type=jax.ShapeDtypeStruct((G2,), jnp.float32),
    mesh=_mesh,
    scratch_types=[
        pltpu.VMEM((EPT,), jnp.int32),
        pltpu.VMEM((CHUNK,), jnp.float32),
        pltpu.VMEM((NPT,), jnp.float32),
        pltpu.VMEM_SHARED((G2,), jnp.float32),
    ],
)
def _sc_degree(dst_hbm, deg_hbm, dst_v, ones_v, zero_v, deg_sh):
    c = lax.axis_index("core")
    s = lax.axis_index("subcore")
    base = c * NP + s * NPT

    @pl.loop(0, CHUNK // 16)
    def _(i):
        ones_v[pl.ds(i * 16, 16)] = jnp.full((16,), 1.0, jnp.float32)

    @pl.loop(0, NPT // 16)
    def _(i):
        zero_v[pl.ds(i * 16, 16)] = jnp.zeros((16,), jnp.float32)

    pltpu.sync_copy(zero_v, deg_sh.at[pl.ds(base, NPT)])
    pltpu.sync_copy(dst_hbm.at[c, s], dst_v)
    plsc.subcore_barrier()

    @pl.loop(0, NCHUNK)
    def _(j):
        pltpu.sync_copy(ones_v, deg_sh.at[dst_v.at[pl.ds(j * CHUNK, CHUNK)]],
                        add=True)

    plsc.subcore_barrier()
    pltpu.sync_copy(deg_sh.at[pl.ds(base, NPT)], deg_hbm.at[pl.ds(base, NPT)])


@functools.partial(
    pl.kernel,
    out_type=(
        jax.ShapeDtypeStruct((G2,), jnp.float32),
        jax.ShapeDtypeStruct((G2, F), jnp.float32),
    ),
    mesh=_mesh,
    scratch_types=[
        pltpu.VMEM((EPT,), jnp.int32),
        pltpu.VMEM((EPT,), jnp.int32),
        pltpu.VMEM((EPT,), jnp.float32),
        pltpu.VMEM((G2,), jnp.float32),
        pltpu.VMEM((CHUNK, F), jnp.float32),
        pltpu.VMEM((CHUNK, F), jnp.float32),
        pltpu.VMEM((NPT, F), jnp.float32),
        pltpu.VMEM((NPT,), jnp.float32),
        pltpu.VMEM_SHARED((G2,), jnp.float32),
        pltpu.VMEM_SHARED((G2, F), jnp.float32),
        pltpu.SemaphoreType.DMA,
        pltpu.SemaphoreType.DMA,
    ],
    compiler_params=_sc_params,
)
def _sc_agg(src_hbm, dst_hbm, dinv_hbm, g_hbm, t_hbm, acc_hbm,
            src_v, dst_v, tval_v, dinv_v, rows0_v, rows1_v, zrows_v, zero_v,
            t_sh, acc_sh, gsem, ssem):
    c = lax.axis_index("core")
    s = lax.axis_index("subcore")
    base = c * NP + s * NPT

    @pl.loop(0, NPT)
    def _(i):
        zrows_v[i, :] = jnp.zeros((F,), jnp.float32)

    @pl.loop(0, NPT // 16)
    def _(i):
        zero_v[pl.ds(i * 16, 16)] = jnp.zeros((16,), jnp.float32)

    pltpu.sync_copy(zrows_v, acc_sh.at[pl.ds(base, NPT)])
    pltpu.sync_copy(zero_v, t_sh.at[pl.ds(base, NPT)])
    pltpu.sync_copy(src_hbm.at[c, s], src_v)
    pltpu.sync_copy(dst_hbm.at[c, s], dst_v)
    pltpu.sync_copy(dinv_hbm, dinv_v)

    @pl.loop(0, EPT // 16)
    def _(k):
        idx = dst_v[pl.ds(k * 16, 16)]
        tval_v[pl.ds(k * 16, 16)] = plsc.load_gather(dinv_v, [idx])

    plsc.subcore_barrier()

    def gather_start(jj, buf):
        pltpu.async_copy(g_hbm.at[src_v.at[pl.ds(jj * CHUNK, CHUNK)]],
                         buf, gsem)

    def gather_wait(buf):
        pltpu.make_async_copy(g_hbm.at[src_v.at[pl.ds(0, CHUNK)]], buf,
                              gsem).wait()

    def process(jj, buf):
        sl = pl.ds(jj * CHUNK, CHUNK)
        desc = pltpu.async_copy(buf, acc_sh.at[dst_v.at[sl]], ssem, add=True)
        pltpu.sync_copy(tval_v.at[sl], t_sh.at[src_v.at[sl]], add=True)
        desc.wait()

    gather_start(0, rows0_v)

    @pl.loop(0, NCHUNK, step=2)
    def _(j):
        gather_wait(rows0_v)
        gather_start(j + 1, rows1_v)
        process(j, rows0_v)
        gather_wait(rows1_v)

        @pl.when(j + 2 < NCHUNK)
        def _():
            gather_start(j + 2, rows0_v)

        process(j + 1, rows1_v)

    plsc.subcore_barrier()
    pltpu.sync_copy(t_sh.at[pl.ds(base, NPT)], t_hbm.at[pl.ds(base, NPT)])
    pltpu.sync_copy(acc_sh.at[pl.ds(base, NPT)], acc_hbm.at[pl.ds(base, NPT)])


def _mm_body(x_ref, w_ref, o_ref):
    o_ref[0] = jnp.dot(x_ref[0], w_ref[0], preferred_element_type=jnp.float32)


def _scale_body(deg_ref, h_ref, dinv_ref, invdeg_ref, g_ref):
    deg = deg_ref[...] + 1.0
    dinv = lax.rsqrt(deg)
    dinv_ref[...] = dinv
    invdeg_ref[...] = 1.0 / deg
    g_ref[...] = h_ref[...] * dinv[..., None]


def _head_body(h_ref, acc_ref, t_ref, dinv_ref, invdeg_ref,
               bpin_ref, blin_ref, wpout_ref, bpout_ref, wlout_ref, blout_ref,
               w1_ref, b1_ref, w2_ref, b2_ref, w3_ref, b3_ref, act_ref, o_ref):
    mask = (lax.broadcasted_iota(jnp.int32, (NP, 1), 0) < N).astype(jnp.float32)

    def pool(gi, b_vec, w_out, b_out):
        dinv = dinv_ref[gi][:, None]
        invdeg = invdeg_ref[gi][:, None]
        out1 = dinv * acc_ref[gi] + invdeg * h_ref[gi] + b_vec
        h1 = jnp.maximum(out1, 0.0)
        cvec = (dinv * t_ref[gi][:, None] + invdeg) * mask
        s_vec = jnp.sum(cvec * h1, axis=0, keepdims=True)
        return jnp.dot(s_vec / float(N), w_out,
                       preferred_element_type=jnp.float32) + b_out

    p = pool(0, bpin_ref[...], wpout_ref[...], bpout_ref[...])
    l = pool(1, blin_ref[...], wlout_ref[...], blout_ref[...])
    fp = jnp.maximum(
        jnp.dot(p, w1_ref[0:50, :], preferred_element_type=jnp.float32)
        + jnp.dot(l, w1_ref[50:100, :], preferred_element_type=jnp.float32)
        + b1_ref[...], 0.0)
    pol = (jnp.dot(fp, w2_ref[0:60, :], preferred_element_type=jnp.float32)
           + jnp.dot(act_ref[...], w2_ref[60:100, :],
                     preferred_element_type=jnp.float32)
           + b2_ref[...])
    o_ref[...] = jnp.dot(jnp.maximum(pol, 0.0), w3_ref[...],
                         preferred_element_type=jnp.float32) + b3_ref[...]


def _prep_edges(ei, graph):
    pad = jnp.full((EP - E,), N, jnp.int32)
    off = graph * NP
    src = jnp.concatenate([ei[0], pad]) + off
    dst = jnp.concatenate([ei[1], pad]) + off
    return src.reshape(NTILE, EPT), dst.reshape(NTILE, EPT)


def kernel(protein_x, protein_edge_index, ligand_x, ligand_edge_index, action,
           W_pin, b_pin, W_pout, b_pout, W_lin, b_lin, W_lout, b_lout,
           W1, b1, W2, b2, W3, b3):
    f32 = jnp.float32
    src_p, dst_p = _prep_edges(protein_edge_index, 0)
    src_l, dst_l = _prep_edges(ligand_edge_index, 1)
    srcs = jnp.stack([src_p, src_l])
    dsts = jnp.stack([dst_p, dst_l])
    xs = jnp.stack([
        jnp.pad(protein_x, ((0, NP - N), (0, 0))),
        jnp.pad(ligand_x, ((0, NP - N), (0, 0))),
    ])
    ws = jnp.stack([W_pin, W_lin])

    hs = pl.pallas_call(
        _mm_body,
        grid=(2, NP // 1024),
        in_specs=[
            pl.BlockSpec((1, 1024, D), lambda g, i: (g, i, 0)),
            pl.BlockSpec((1, D, F), lambda g, i: (g, 0, 0)),
        ],
        out_specs=pl.BlockSpec((1, 1024, F), lambda g, i: (g, i, 0)),
        out_shape=jax.ShapeDtypeStruct((2, NP, F), f32),
    )(xs, ws)

    deg = _sc_degree(dsts)

    dinv, invdeg, g = pl.pallas_call(
        _scale_body,
        out_shape=[jax.ShapeDtypeStruct((2, NP), f32),
                   jax.ShapeDtypeStruct((2, NP), f32),
                   jax.ShapeDtypeStruct((2, NP, F), f32)],
    )(deg.reshape(2, NP), hs)

    t, acc = _sc_agg(srcs, dsts, dinv.reshape(G2), g.reshape(G2, F))

    out = pl.pallas_call(
        _head_body,
        out_shape=jax.ShapeDtypeStruct((1, 1), f32),
    )(hs, acc.reshape(2, NP, F), t.reshape(2, NP), dinv, invdeg,
      b_pin.reshape(1, F), b_lin.reshape(1, F), W_pout, b_pout.reshape(1, 50),
      W_lout, b_lout.reshape(1, 50), W1, b1.reshape(1, 60), W2,
      b2.reshape(1, 10), W3, b3.reshape(1, 1), action)
    return out


# speedup vs baseline: 64.6266x; 1.0546x over previous
import functools

import jax
import jax.numpy as jnp
from jax import lax
from jax.experimental import pallas as pl
from jax.experimental.pallas import tpu as pltpu
from jax.experimental.pallas import tpu_sc as plsc

N = 10000
NP = 10240
E = 320000
D = 128
F = 16
NTILE = 16
CHUNK = 512
NCHUNK = 40
EPT = NCHUNK * CHUNK
EP = EPT * NTILE
NPT = NP // NTILE
G2 = 2 * NP

_mesh = plsc.VectorSubcoreMesh(core_axis_name="core", subcore_axis_name="subcore")

_sc_params = pltpu.CompilerParams(
    needs_layout_passes=False, use_tc_tiling_on_sc=False)


@functools.partial(
    pl.kernel,
    out_type=(
        jax.ShapeDtypeStruct((G2,), jnp.float32),
        jax.ShapeDtypeStruct((G2,), jnp.float32),
        jax.ShapeDtypeStruct((G2, F), jnp.float32),
        jax.ShapeDtypeStruct((G2, F), jnp.float32),
    ),
    mesh=_mesh,
    scratch_types=[
        pltpu.VMEM((EPT,), jnp.int32),
        pltpu.VMEM((EPT,), jnp.int32),
        pltpu.VMEM((CHUNK,), jnp.float32),
        pltpu.VMEM((CHUNK,), jnp.float32),
        pltpu.VMEM((CHUNK, F), jnp.float32),
        pltpu.VMEM((CHUNK, F), jnp.float32),
        pltpu.VMEM((NPT, F), jnp.float32),
        pltpu.VMEM((NPT,), jnp.float32),
        pltpu.VMEM((NPT,), jnp.float32),
        pltpu.VMEM((CHUNK,), jnp.float32),
        pltpu.VMEM_SHARED((G2,), jnp.float32),
        pltpu.VMEM_SHARED((G2,), jnp.float32),
        pltpu.VMEM_SHARED((G2, F), jnp.float32),
        pltpu.SemaphoreType.DMA,
        pltpu.SemaphoreType.DMA,
        pltpu.SemaphoreType.DMA,
    ],
    compiler_params=_sc_params,
)
def _sc_mega(src_hbm, dst_hbm, h_hbm, deg_hbm, t_hbm, acc_hbm, g_hbm,
             src_v, dst_v, tv0_v, tv1_v, rows0_v, rows1_v, hrows_v,
             degn_v, dinvn_v, ones_v, t_sh, dinv_sh, acc_sh,
             gsem, ssem, tsem):
    c = lax.axis_index("core")
    s = lax.axis_index("subcore")
    base = c * NP + s * NPT

    @pl.loop(0, CHUNK // 16)
    def _(i):
        ones_v[pl.ds(i * 16, 16)] = jnp.full((16,), 1.0, jnp.float32)

    @pl.loop(0, NPT)
    def _(i):
        hrows_v[i, :] = jnp.zeros((F,), jnp.float32)

    @pl.loop(0, NPT // 16)
    def _(i):
        degn_v[pl.ds(i * 16, 16)] = jnp.zeros((16,), jnp.float32)

    pltpu.sync_copy(degn_v, t_sh.at[pl.ds(base, NPT)])
    pltpu.sync_copy(hrows_v, acc_sh.at[pl.ds(base, NPT)])
    pltpu.sync_copy(src_hbm.at[c, s], src_v)
    pltpu.sync_copy(dst_hbm.at[c, s], dst_v)
    plsc.subcore_barrier()

    @pl.loop(0, NCHUNK)
    def _(j):
        pltpu.sync_copy(ones_v, t_sh.at[dst_v.at[pl.ds(j * CHUNK, CHUNK)]],
                        add=True)

    plsc.subcore_barrier()

    pltpu.sync_copy(t_sh.at[pl.ds(base, NPT)], degn_v)
    pltpu.sync_copy(degn_v, deg_hbm.at[pl.ds(base, NPT)])

    @pl.loop(0, NPT // 16)
    def _(i):
        x = degn_v[pl.ds(i * 16, 16)] + 1.0
        bits = lax.bitcast_convert_type(x, jnp.int32)
        y = lax.bitcast_convert_type(0x5F3759DF - (bits >> 1), jnp.float32)
        half_x = 0.5 * x
        y = y * (1.5 - half_x * y * y)
        y = y * (1.5 - half_x * y * y)
        y = y * (1.5 - half_x * y * y)
        dinvn_v[pl.ds(i * 16, 16)] = y
        degn_v[pl.ds(i * 16, 16)] = jnp.zeros((16,), jnp.float32)

    pltpu.sync_copy(dinvn_v, dinv_sh.at[pl.ds(base, NPT)])
    pltpu.sync_copy(degn_v, t_sh.at[pl.ds(base, NPT)])

    pltpu.sync_copy(h_hbm.at[pl.ds(base, NPT)], hrows_v)

    @pl.loop(0, NPT // 16)
    def _(i):
        dv = dinvn_v[pl.ds(i * 16, 16)]
        for k in range(16):
            hrows_v[i * 16 + k, :] = hrows_v[i * 16 + k, :] * dv[k]

    pltpu.sync_copy(hrows_v, g_hbm.at[pl.ds(base, NPT)])
    plsc.subcore_barrier()

    def gather_start(jj, buf, tbuf):
        sl = pl.ds(jj * CHUNK, CHUNK)
        pltpu.async_copy(g_hbm.at[src_v.at[sl]], buf, gsem)
        pltpu.async_copy(dinv_sh.at[dst_v.at[sl]], tbuf, tsem)

    def gather_wait(buf, tbuf):
        pltpu.make_async_copy(g_hbm.at[src_v.at[pl.ds(0, CHUNK)]], buf,
                              gsem).wait()
        pltpu.make_async_copy(dinv_sh.at[dst_v.at[pl.ds(0, CHUNK)]], tbuf,
                              tsem).wait()

    def process(jj, buf, tbuf):
        sl = pl.ds(jj * CHUNK, CHUNK)
        desc = pltpu.async_copy(buf, acc_sh.at[dst_v.at[sl]], ssem, add=True)
        pltpu.sync_copy(tbuf, t_sh.at[src_v.at[sl]], add=True)
        desc.wait()

    gather_start(0, rows0_v, tv0_v)

    @pl.loop(0, NCHUNK, step=2)
    def _(j):
        gather_wait(rows0_v, tv0_v)
        gather_start(j + 1, rows1_v, tv1_v)
        process(j, rows0_v, tv0_v)
        gather_wait(rows1_v, tv1_v)

        @pl.when(j + 2 < NCHUNK)
        def _():
            gather_start(j + 2, rows0_v, tv0_v)

        process(j + 1, rows1_v, tv1_v)

    plsc.subcore_barrier()
    pltpu.sync_copy(t_sh.at[pl.ds(base, NPT)], t_hbm.at[pl.ds(base, NPT)])
    pltpu.sync_copy(acc_sh.at[pl.ds(base, NPT)], acc_hbm.at[pl.ds(base, NPT)])


def _mm_body(x_ref, w_ref, o_ref):
    o_ref[0] = jnp.dot(x_ref[0], w_ref[0], preferred_element_type=jnp.float32)


def _head_body(h_ref, acc_ref, t_ref, deg_ref,
               bpin_ref, blin_ref, wpout_ref, bpout_ref, wlout_ref, blout_ref,
               w1_ref, b1_ref, w2_ref, b2_ref, w3_ref, b3_ref, act_ref, o_ref):
    mask = (lax.broadcasted_iota(jnp.int32, (NP, 1), 0) < N).astype(jnp.float32)

    def pool(gi, b_vec, w_out, b_out):
        deg = deg_ref[gi][:, None] + 1.0
        dinv = lax.rsqrt(deg)
        invdeg = 1.0 / deg
        out1 = dinv * acc_ref[gi] + invdeg * h_ref[gi] + b_vec
        h1 = jnp.maximum(out1, 0.0)
        cvec = (dinv * t_ref[gi][:, None] + invdeg) * mask
        s_vec = jnp.sum(cvec * h1, axis=0, keepdims=True)
        return jnp.dot(s_vec / float(N), w_out,
                       preferred_element_type=jnp.float32) + b_out

    p = pool(0, bpin_ref[...], wpout_ref[...], bpout_ref[...])
    l = pool(1, blin_ref[...], wlout_ref[...], blout_ref[...])
    fp = jnp.maximum(
        jnp.dot(p, w1_ref[0:50, :], preferred_element_type=jnp.float32)
        + jnp.dot(l, w1_ref[50:100, :], preferred_element_type=jnp.float32)
        + b1_ref[...], 0.0)
    pol = (jnp.dot(fp, w2_ref[0:60, :], preferred_element_type=jnp.float32)
           + jnp.dot(act_ref[...], w2_ref[60:100, :],
                     preferred_element_type=jnp.float32)
           + b2_ref[...])
    o_ref[...] = jnp.dot(jnp.maximum(pol, 0.0), w3_ref[...],
                         preferred_element_type=jnp.float32) + b3_ref[...]


def _prep_edges(ei, graph):
    pad = jnp.full((EP - E,), N, jnp.int32)
    off = graph * NP
    src = jnp.concatenate([ei[0], pad]) + off
    dst = jnp.concatenate([ei[1], pad]) + off
    return src.reshape(NTILE, EPT), dst.reshape(NTILE, EPT)


def kernel(protein_x, protein_edge_index, ligand_x, ligand_edge_index, action,
           W_pin, b_pin, W_pout, b_pout, W_lin, b_lin, W_lout, b_lout,
           W1, b1, W2, b2, W3, b3):
    f32 = jnp.float32
    src_p, dst_p = _prep_edges(protein_edge_index, 0)
    src_l, dst_l = _prep_edges(ligand_edge_index, 1)
    srcs = jnp.stack([src_p, src_l])
    dsts = jnp.stack([dst_p, dst_l])
    xs = jnp.stack([
        jnp.pad(protein_x, ((0, NP - N), (0, 0))),
        jnp.pad(ligand_x, ((0, NP - N), (0, 0))),
    ])
    ws = jnp.stack([W_pin, W_lin])

    hs = pl.pallas_call(
        _mm_body,
        grid=(2, NP // 1024),
        in_specs=[
            pl.BlockSpec((1, 1024, D), lambda g, i: (g, i, 0)),
            pl.BlockSpec((1, D, F), lambda g, i: (g, 0, 0)),
        ],
        out_specs=pl.BlockSpec((1, 1024, F), lambda g, i: (g, i, 0)),
        out_shape=jax.ShapeDtypeStruct((2, NP, F), f32),
    )(xs, ws)

    deg, t, acc, _g = _sc_mega(srcs, dsts, hs.reshape(G2, F))

    out = pl.pallas_call(
        _head_body,
        out_shape=jax.ShapeDtypeStruct((1, 1), f32),
    )(hs, acc.reshape(2, NP, F), t.reshape(2, NP), deg.reshape(2, NP),
      b_pin.reshape(1, F), b_lin.reshape(1, F), W_pout, b_pout.reshape(1, 50),
      W_lout, b_lout.reshape(1, 50), W1, b1.reshape(1, 60), W2,
      b2.reshape(1, 10), W3, b3.reshape(1, 1), action)
    return out


# speedup vs baseline: 84.2172x; 1.3031x over previous
import functools

import jax
import jax.numpy as jnp
from jax import lax
from jax.experimental import pallas as pl
from jax.experimental.pallas import tpu as pltpu
from jax.experimental.pallas import tpu_sc as plsc

N = 10000
NP = 10240
E = 320000
D = 128
F = 16
NTILE = 16
CHUNK = 512
NCHUNK = 40
EPT = NCHUNK * CHUNK
EP = EPT * NTILE
NPT = NP // NTILE
G2 = 2 * NP

_mesh = plsc.VectorSubcoreMesh(core_axis_name="core", subcore_axis_name="subcore")

_sc_params = pltpu.CompilerParams(
    needs_layout_passes=False, use_tc_tiling_on_sc=False)


@functools.partial(
    pl.kernel,
    out_type=(
        jax.ShapeDtypeStruct((G2,), jnp.float32),
        jax.ShapeDtypeStruct((G2,), jnp.float32),
        jax.ShapeDtypeStruct((G2, F), jnp.float32),
        jax.ShapeDtypeStruct((2, NP, F), jnp.float32),
    ),
    mesh=_mesh,
    scratch_types=[
        pltpu.VMEM((EPT,), jnp.int32),
        pltpu.VMEM((EPT,), jnp.int32),
        pltpu.VMEM((CHUNK,), jnp.float32),
        pltpu.VMEM((CHUNK,), jnp.float32),
        pltpu.VMEM((CHUNK, F), jnp.float32),
        pltpu.VMEM((CHUNK, F), jnp.float32),
        pltpu.VMEM((NPT, F), jnp.float32),
        pltpu.VMEM((NPT,), jnp.float32),
        pltpu.VMEM((NPT,), jnp.float32),
        pltpu.VMEM((CHUNK,), jnp.float32),
        pltpu.VMEM_SHARED((NP,), jnp.float32),
        pltpu.VMEM_SHARED((NP,), jnp.float32),
        pltpu.VMEM_SHARED((NP, F), jnp.float32),
        pltpu.SemaphoreType.DMA,
        pltpu.SemaphoreType.DMA,
        pltpu.SemaphoreType.DMA,
    ],
    compiler_params=_sc_params,
)
def _sc_mega(ei_hbm, h_hbm, deg_hbm, t_hbm, acc_hbm, g_hbm,
             src_v, dst_v, tv0_v, tv1_v, rows0_v, rows1_v, hrows_v,
             degn_v, dinvn_v, ones_v, t_sh, dinv_sh, acc_sh,
             gsem, ssem, tsem):
    c = lax.axis_index("core")
    s = lax.axis_index("subcore")
    base = c * NP + s * NPT
    lbase = s * NPT

    @pl.loop(0, CHUNK // 16)
    def _(i):
        ones_v[pl.ds(i * 16, 16)] = jnp.full((16,), 1.0, jnp.float32)

    @pl.loop(0, NPT)
    def _(i):
        hrows_v[i, :] = jnp.zeros((F,), jnp.float32)

    @pl.loop(0, NPT // 16)
    def _(i):
        degn_v[pl.ds(i * 16, 16)] = jnp.zeros((16,), jnp.float32)

    pltpu.sync_copy(degn_v, t_sh.at[pl.ds(lbase, NPT)])
    pltpu.sync_copy(hrows_v, acc_sh.at[pl.ds(lbase, NPT)])
    pltpu.sync_copy(ei_hbm.at[c, 0, pl.ds(s * EPT, EPT)], src_v)
    pltpu.sync_copy(ei_hbm.at[c, 1, pl.ds(s * EPT, EPT)], dst_v)
    plsc.subcore_barrier()

    @pl.loop(0, NCHUNK)
    def _(j):
        pltpu.sync_copy(ones_v, t_sh.at[dst_v.at[pl.ds(j * CHUNK, CHUNK)]],
                        add=True)

    plsc.subcore_barrier()

    pltpu.sync_copy(t_sh.at[pl.ds(lbase, NPT)], degn_v)
    pltpu.sync_copy(degn_v, deg_hbm.at[pl.ds(base, NPT)])

    @pl.loop(0, NPT // 16)
    def _(i):
        x = degn_v[pl.ds(i * 16, 16)] + 1.0
        bits = lax.bitcast_convert_type(x, jnp.int32)
        y = lax.bitcast_convert_type(0x5F3759DF - (bits >> 1), jnp.float32)
        half_x = 0.5 * x
        y = y * (1.5 - half_x * y * y)
        y = y * (1.5 - half_x * y * y)
        y = y * (1.5 - half_x * y * y)
        dinvn_v[pl.ds(i * 16, 16)] = y
        degn_v[pl.ds(i * 16, 16)] = jnp.zeros((16,), jnp.float32)

    pltpu.sync_copy(dinvn_v, dinv_sh.at[pl.ds(lbase, NPT)])
    pltpu.sync_copy(degn_v, t_sh.at[pl.ds(lbase, NPT)])

    pltpu.sync_copy(h_hbm.at[c, pl.ds(lbase, NPT)], hrows_v)

    @pl.loop(0, NPT // 16)
    def _(i):
        dv = dinvn_v[pl.ds(i * 16, 16)]
        for k in range(16):
            hrows_v[i * 16 + k, :] = hrows_v[i * 16 + k, :] * dv[k]

    pltpu.sync_copy(hrows_v, g_hbm.at[c, pl.ds(lbase, NPT)])
    plsc.subcore_barrier()

    def gather_start(jj, buf, tbuf):
        sl = pl.ds(jj * CHUNK, CHUNK)
        pltpu.async_copy(g_hbm.at[c].at[src_v.at[sl]], buf, gsem)
        pltpu.async_copy(dinv_sh.at[dst_v.at[sl]], tbuf, tsem)

    def gather_wait(buf, tbuf):
        pltpu.make_async_copy(g_hbm.at[c].at[src_v.at[pl.ds(0, CHUNK)]], buf,
                              gsem).wait()
        pltpu.make_async_copy(dinv_sh.at[dst_v.at[pl.ds(0, CHUNK)]], tbuf,
                              tsem).wait()

    def process(jj, buf, tbuf):
        sl = pl.ds(jj * CHUNK, CHUNK)
        desc = pltpu.async_copy(buf, acc_sh.at[dst_v.at[sl]], ssem, add=True)
        pltpu.sync_copy(tbuf, t_sh.at[src_v.at[sl]], add=True)
        desc.wait()

    gather_start(0, rows0_v, tv0_v)

    @pl.loop(0, NCHUNK, step=2)
    def _(j):
        gather_wait(rows0_v, tv0_v)
        gather_start(j + 1, rows1_v, tv1_v)
        process(j, rows0_v, tv0_v)
        gather_wait(rows1_v, tv1_v)

        @pl.when(j + 2 < NCHUNK)
        def _():
            gather_start(j + 2, rows0_v, tv0_v)

        process(j + 1, rows1_v, tv1_v)

    plsc.subcore_barrier()
    pltpu.sync_copy(t_sh.at[pl.ds(lbase, NPT)], t_hbm.at[pl.ds(base, NPT)])
    pltpu.sync_copy(acc_sh.at[pl.ds(lbase, NPT)],
                    acc_hbm.at[pl.ds(base, NPT)])


def _mm_body(xp_ref, xl_ref, wp_ref, wl_ref, o_ref):
    zero = jnp.zeros((NP - N, F), jnp.float32)
    o_ref[0, 0:N, :] = jnp.dot(xp_ref[...], wp_ref[...],
                               preferred_element_type=jnp.float32)
    o_ref[0, N:NP, :] = zero
    o_ref[1, 0:N, :] = jnp.dot(xl_ref[...], wl_ref[...],
                               preferred_element_type=jnp.float32)
    o_ref[1, N:NP, :] = zero


def _head_body(h_ref, acc_ref, t_ref, deg_ref,
               bpin_ref, blin_ref, wpout_ref, bpout_ref, wlout_ref, blout_ref,
               w1_ref, b1_ref, w2_ref, b2_ref, w3_ref, b3_ref, act_ref, o_ref):
    mask = (lax.broadcasted_iota(jnp.int32, (NP, 1), 0) < N).astype(jnp.float32)

    def pool(gi, b_vec, w_out, b_out):
        deg = deg_ref[gi][:, None] + 1.0
        dinv = lax.rsqrt(deg)
        invdeg = 1.0 / deg
        out1 = dinv * acc_ref[gi] + invdeg * h_ref[gi] + b_vec
        h1 = jnp.maximum(out1, 0.0)
        cvec = (dinv * t_ref[gi][:, None] + invdeg) * mask
        s_vec = jnp.sum(cvec * h1, axis=0, keepdims=True)
        return jnp.dot(s_vec / float(N), w_out,
                       preferred_element_type=jnp.float32) + b_out

    p = pool(0, bpin_ref[...], wpout_ref[...], bpout_ref[...])
    l = pool(1, blin_ref[...], wlout_ref[...], blout_ref[...])
    fp = jnp.maximum(
        jnp.dot(p, w1_ref[0:50, :], preferred_element_type=jnp.float32)
        + jnp.dot(l, w1_ref[50:100, :], preferred_element_type=jnp.float32)
        + b1_ref[...], 0.0)
    pol = (jnp.dot(fp, w2_ref[0:60, :], preferred_element_type=jnp.float32)
           + jnp.dot(act_ref[...], w2_ref[60:100, :],
                     preferred_element_type=jnp.float32)
           + b2_ref[...])
    o_ref[...] = jnp.dot(jnp.maximum(pol, 0.0), w3_ref[...],
                         preferred_element_type=jnp.float32) + b3_ref[...]


def kernel(protein_x, protein_edge_index, ligand_x, ligand_edge_index, action,
           W_pin, b_pin, W_pout, b_pout, W_lin, b_lin, W_lout, b_lout,
           W1, b1, W2, b2, W3, b3):
    f32 = jnp.float32
    eis = jnp.stack([
        jnp.pad(protein_edge_index, ((0, 0), (0, EP - E)), constant_values=N),
        jnp.pad(ligand_edge_index, ((0, 0), (0, EP - E)), constant_values=N),
    ])

    hs = pl.pallas_call(
        _mm_body,
        out_shape=jax.ShapeDtypeStruct((2, NP, F), f32),
    )(protein_x, ligand_x, W_pin, W_lin)

    deg, t, acc, _g = _sc_mega(eis, hs)

    out = pl.pallas_call(
        _head_body,
        out_shape=jax.ShapeDtypeStruct((1, 1), f32),
    )(hs, acc.reshape(2, NP, F), t.reshape(2, NP), deg.reshape(2, NP),
      b_pin.reshape(1, F), b_lin.reshape(1, F), W_pout, b_pout.reshape(1, 50),
      W_lout, b_lout.reshape(1, 50), W1, b1.reshape(1, 60), W2,
      b2.reshape(1, 10), W3, b3.reshape(1, 1), action)
    return out


# speedup vs baseline: 86.9318x; 1.0322x over previous
import functools

import jax
import jax.numpy as jnp
from jax import lax
from jax.experimental import pallas as pl
from jax.experimental.pallas import tpu as pltpu
from jax.experimental.pallas import tpu_sc as plsc

N = 10000
NP = 10240
E = 320000
D = 128
F = 16
NTILE = 16
CHUNK = 512
NCHUNK = 40
EPT = NCHUNK * CHUNK
EP = EPT * NTILE
NPT = NP // NTILE
G2 = 2 * NP

_mesh = plsc.VectorSubcoreMesh(core_axis_name="core", subcore_axis_name="subcore")

_sc_params = pltpu.CompilerParams(
    needs_layout_passes=False, use_tc_tiling_on_sc=False)


@functools.partial(
    pl.kernel,
    out_type=(
        jax.ShapeDtypeStruct((2, F), jnp.float32),
        jax.ShapeDtypeStruct((2, NP, F), jnp.float32),
    ),
    mesh=_mesh,
    scratch_types=[
        pltpu.VMEM((EPT,), jnp.int32),
        pltpu.VMEM((EPT,), jnp.int32),
        pltpu.VMEM((CHUNK,), jnp.float32),
        pltpu.VMEM((CHUNK,), jnp.float32),
        pltpu.VMEM((CHUNK, F), jnp.float32),
        pltpu.VMEM((CHUNK, F), jnp.float32),
        pltpu.VMEM((NPT, F), jnp.float32),
        pltpu.VMEM((NPT, F), jnp.float32),
        pltpu.VMEM((NPT,), jnp.float32),
        pltpu.VMEM((NPT,), jnp.float32),
        pltpu.VMEM((CHUNK,), jnp.float32),
        pltpu.VMEM((F, F), jnp.float32),
        pltpu.VMEM_SHARED((NP,), jnp.float32),
        pltpu.VMEM_SHARED((NP,), jnp.float32),
        pltpu.VMEM_SHARED((NP, F), jnp.float32),
        pltpu.VMEM_SHARED((F, F), jnp.float32),
        pltpu.SemaphoreType.DMA,
        pltpu.SemaphoreType.DMA,
        pltpu.SemaphoreType.DMA,
    ],
    compiler_params=_sc_params,
)
def _sc_mega(ei_hbm, h_hbm, b_hbm, s_hbm, g_hbm,
             src_v, dst_v, tv0_v, tv1_v, rows0_v, rows1_v, hrows_v, hn_v,
             degn_v, dinvn_v, ones_v, psum_v, t_sh, dinv_sh, acc_sh, part_sh,
             gsem, ssem, tsem):
    c = lax.axis_index("core")
    s = lax.axis_index("subcore")
    base = c * NP + s * NPT
    lbase = s * NPT

    @pl.loop(0, CHUNK // 16)
    def _(i):
        ones_v[pl.ds(i * 16, 16)] = jnp.full((16,), 1.0, jnp.float32)

    @pl.loop(0, NPT)
    def _(i):
        hrows_v[i, :] = jnp.zeros((F,), jnp.float32)

    @pl.loop(0, NPT // 16)
    def _(i):
        degn_v[pl.ds(i * 16, 16)] = jnp.zeros((16,), jnp.float32)

    pltpu.sync_copy(degn_v, t_sh.at[pl.ds(lbase, NPT)])
    pltpu.sync_copy(hrows_v, acc_sh.at[pl.ds(lbase, NPT)])
    pltpu.sync_copy(ei_hbm.at[c, 0, pl.ds(s * EPT, EPT)], src_v)
    pltpu.sync_copy(ei_hbm.at[c, 1, pl.ds(s * EPT, EPT)], dst_v)
    plsc.subcore_barrier()

    @pl.loop(0, NCHUNK)
    def _(j):
        pltpu.sync_copy(ones_v, t_sh.at[dst_v.at[pl.ds(j * CHUNK, CHUNK)]],
                        add=True)

    plsc.subcore_barrier()

    pltpu.sync_copy(t_sh.at[pl.ds(lbase, NPT)], degn_v)

    @pl.loop(0, NPT // 16)
    def _(i):
        x = degn_v[pl.ds(i * 16, 16)] + 1.0
        bits = lax.bitcast_convert_type(x, jnp.int32)
        y = lax.bitcast_convert_type(0x5F3759DF - (bits >> 1), jnp.float32)
        half_x = 0.5 * x
        y = y * (1.5 - half_x * y * y)
        y = y * (1.5 - half_x * y * y)
        y = y * (1.5 - half_x * y * y)
        dinvn_v[pl.ds(i * 16, 16)] = y
        degn_v[pl.ds(i * 16, 16)] = jnp.zeros((16,), jnp.float32)

    pltpu.sync_copy(dinvn_v, dinv_sh.at[pl.ds(lbase, NPT)])
    pltpu.sync_copy(degn_v, t_sh.at[pl.ds(lbase, NPT)])

    pltpu.sync_copy(h_hbm.at[pl.ds(base, NPT)], hn_v)

    @pl.loop(0, NPT // 16)
    def _(i):
        dv = dinvn_v[pl.ds(i * 16, 16)]
        for k in range(16):
            hrows_v[i * 16 + k, :] = hn_v[i * 16 + k, :] * dv[k]

    pltpu.sync_copy(hrows_v, g_hbm.at[c, pl.ds(lbase, NPT)])
    plsc.subcore_barrier()

    def gather_start(jj, buf, tbuf):
        sl = pl.ds(jj * CHUNK, CHUNK)
        pltpu.async_copy(g_hbm.at[c].at[src_v.at[sl]], buf, gsem)
        pltpu.async_copy(dinv_sh.at[dst_v.at[sl]], tbuf, tsem)

    def gather_wait(buf, tbuf):
        pltpu.make_async_copy(g_hbm.at[c].at[src_v.at[pl.ds(0, CHUNK)]], buf,
                              gsem).wait()
        pltpu.make_async_copy(dinv_sh.at[dst_v.at[pl.ds(0, CHUNK)]], tbuf,
                              tsem).wait()

    def process(jj, buf, tbuf):
        sl = pl.ds(jj * CHUNK, CHUNK)
        desc = pltpu.async_copy(buf, acc_sh.at[dst_v.at[sl]], ssem, add=True)
        pltpu.sync_copy(tbuf, t_sh.at[src_v.at[sl]], add=True)
        desc.wait()

    gather_start(0, rows0_v, tv0_v)

    @pl.loop(0, NCHUNK, step=2)
    def _(j):
        gather_wait(rows0_v, tv0_v)
        gather_start(j + 1, rows1_v, tv1_v)
        process(j, rows0_v, tv0_v)
        gather_wait(rows1_v, tv1_v)

        @pl.when(j + 2 < NCHUNK)
        def _():
            gather_start(j + 2, rows0_v, tv0_v)

        process(j + 1, rows1_v, tv1_v)

    plsc.subcore_barrier()

    pltpu.sync_copy(acc_sh.at[pl.ds(lbase, NPT)], hrows_v)
    pltpu.sync_copy(t_sh.at[pl.ds(lbase, NPT)], degn_v)
    pltpu.sync_copy(b_hbm.at[c], psum_v.at[0])
    bvec = psum_v[0, :]

    def body(i, s_acc):
        dv = dinvn_v[pl.ds(i * 16, 16)]
        inv = dv * dv
        node = lbase + i * 16 + lax.iota(jnp.int32, 16)
        cw = jnp.where(node < N, dv * degn_v[pl.ds(i * 16, 16)] + inv, 0.0)
        for k in range(16):
            h1 = jnp.maximum(
                dv[k] * hrows_v[i * 16 + k, :] + inv[k] * hn_v[i * 16 + k, :]
                + bvec, 0.0)
            s_acc = s_acc + cw[k] * h1
        return s_acc

    s_part = pl.loop(0, NPT // 16,
                     init_carry=jnp.zeros((16,), jnp.float32))(body)
    psum_v[1, :] = s_part
    pltpu.sync_copy(psum_v.at[1], part_sh.at[s])
    plsc.subcore_barrier()

    @pl.when(s == 0)
    def _():
        pltpu.sync_copy(part_sh, psum_v)
        tot = psum_v[0, :]
        for k in range(1, 16):
            tot = tot + psum_v[k, :]
        psum_v[0, :] = tot
        pltpu.sync_copy(psum_v.at[0], s_hbm.at[c])


def _mm_body(xp_ref, xl_ref, wp_ref, wl_ref, o_ref):
    zero = jnp.zeros((NP - N, F), jnp.float32)
    o_ref[0, 0:N, :] = jnp.dot(xp_ref[...], wp_ref[...],
                               preferred_element_type=jnp.float32)
    o_ref[0, N:NP, :] = zero
    o_ref[1, 0:N, :] = jnp.dot(xl_ref[...], wl_ref[...],
                               preferred_element_type=jnp.float32)
    o_ref[1, N:NP, :] = zero


def _head_body(s_ref, wpout_ref, bpout_ref, wlout_ref, blout_ref,
               w1_ref, b1_ref, w2_ref, b2_ref, w3_ref, b3_ref, act_ref,
               o_ref):
    p = jnp.dot(s_ref[0:1, :] / float(N), wpout_ref[...],
                preferred_element_type=jnp.float32) + bpout_ref[...]
    l = jnp.dot(s_ref[1:2, :] / float(N), wlout_ref[...],
                preferred_element_type=jnp.float32) + blout_ref[...]
    fp = jnp.maximum(
        jnp.dot(p, w1_ref[0:50, :], preferred_element_type=jnp.float32)
        + jnp.dot(l, w1_ref[50:100, :], preferred_element_type=jnp.float32)
        + b1_ref[...], 0.0)
    pol = (jnp.dot(fp, w2_ref[0:60, :], preferred_element_type=jnp.float32)
           + jnp.dot(act_ref[...], w2_ref[60:100, :],
                     preferred_element_type=jnp.float32)
           + b2_ref[...])
    o_ref[...] = jnp.dot(jnp.maximum(pol, 0.0), w3_ref[...],
                         preferred_element_type=jnp.float32) + b3_ref[...]


def kernel(protein_x, protein_edge_index, ligand_x, ligand_edge_index, action,
           W_pin, b_pin, W_pout, b_pout, W_lin, b_lin, W_lout, b_lout,
           W1, b1, W2, b2, W3, b3):
    f32 = jnp.float32
    eis = jnp.stack([
        jnp.pad(protein_edge_index, ((0, 0), (0, EP - E)), constant_values=N),
        jnp.pad(ligand_edge_index, ((0, 0), (0, EP - E)), constant_values=N),
    ])
    bs = jnp.stack([b_pin, b_lin])

    hs = pl.pallas_call(
        _mm_body,
        out_shape=jax.ShapeDtypeStruct((2, NP, F), f32),
    )(protein_x, ligand_x, W_pin, W_lin)

    s_vec, _g = _sc_mega(eis, hs.reshape(G2, F), bs)

    out = pl.pallas_call(
        _head_body,
        out_shape=jax.ShapeDtypeStruct((1, 1), f32),
    )(s_vec, W_pout, b_pout.reshape(1, 50), W_lout, b_lout.reshape(1, 50),
      W1, b1.reshape(1, 60), W2, b2.reshape(1, 10), W3, b3.reshape(1, 1),
      action)
    return out


# speedup vs baseline: 92.5408x; 1.0645x over previous
import functools

import jax
import jax.numpy as jnp
from jax import lax
from jax.experimental import pallas as pl
from jax.experimental.pallas import tpu as pltpu
from jax.experimental.pallas import tpu_sc as plsc

N = 10000
NP = 10240
E = 320000
D = 128
F = 16
NTILE = 16
CHUNK = 512
NCHUNK = 40
EPT = NCHUNK * CHUNK
EP = EPT * NTILE
NPT = NP // NTILE
G2 = 2 * NP

_mesh = plsc.VectorSubcoreMesh(core_axis_name="core", subcore_axis_name="subcore")

_sc_params = pltpu.CompilerParams(
    needs_layout_passes=False, use_tc_tiling_on_sc=False)


@functools.partial(
    pl.kernel,
    out_type=(
        jax.ShapeDtypeStruct((2, F), jnp.float32),
        jax.ShapeDtypeStruct((2, NP, F), jnp.float32),
    ),
    mesh=_mesh,
    scratch_types=[
        pltpu.VMEM((EPT,), jnp.int32),
        pltpu.VMEM((EPT,), jnp.int32),
        pltpu.VMEM((CHUNK,), jnp.float32),
        pltpu.VMEM((CHUNK,), jnp.float32),
        pltpu.VMEM((CHUNK, F), jnp.float32),
        pltpu.VMEM((CHUNK, F), jnp.float32),
        pltpu.VMEM((NPT, F), jnp.float32),
        pltpu.VMEM((NPT, F), jnp.float32),
        pltpu.VMEM((NPT // 8, 128), jnp.float32),
        pltpu.VMEM((NPT,), jnp.float32),
        pltpu.VMEM((NPT,), jnp.float32),
        pltpu.VMEM((CHUNK,), jnp.float32),
        pltpu.VMEM((F, F), jnp.float32),
        pltpu.VMEM_SHARED((NP,), jnp.float32),
        pltpu.VMEM_SHARED((NP,), jnp.float32),
        pltpu.VMEM_SHARED((NP, F), jnp.float32),
        pltpu.VMEM_SHARED((F, F), jnp.float32),
        pltpu.SemaphoreType.DMA,
        pltpu.SemaphoreType.DMA,
        pltpu.SemaphoreType.DMA,
    ],
    compiler_params=_sc_params,
)
def _sc_mega(ei_hbm, h_hbm, b_hbm, s_hbm, g_hbm,
             src_v, dst_v, tv0_v, tv1_v, rows0_v, rows1_v, hrows_v, hn_v,
             hbuf_v, degn_v, dinvn_v, ones_v, psum_v, t_sh, dinv_sh, acc_sh,
             part_sh, gsem, ssem, tsem):
    c = lax.axis_index("core")
    s = lax.axis_index("subcore")
    base = c * NP + s * NPT
    lbase = s * NPT

    @pl.loop(0, CHUNK // 16)
    def _(i):
        ones_v[pl.ds(i * 16, 16)] = jnp.full((16,), 1.0, jnp.float32)

    @pl.loop(0, NPT)
    def _(i):
        hrows_v[i, :] = jnp.zeros((F,), jnp.float32)

    @pl.loop(0, NPT // 16)
    def _(i):
        degn_v[pl.ds(i * 16, 16)] = jnp.zeros((16,), jnp.float32)

    pltpu.sync_copy(degn_v, t_sh.at[pl.ds(lbase, NPT)])
    pltpu.sync_copy(hrows_v, acc_sh.at[pl.ds(lbase, NPT)])
    pltpu.sync_copy(ei_hbm.at[c, 0, pl.ds(s * EPT, EPT)], src_v)
    pltpu.sync_copy(ei_hbm.at[c, 1, pl.ds(s * EPT, EPT)], dst_v)
    plsc.subcore_barrier()

    @pl.loop(0, NCHUNK)
    def _(j):
        pltpu.async_copy(ones_v, t_sh.at[dst_v.at[pl.ds(j * CHUNK, CHUNK)]],
                         ssem, add=True)

    @pl.loop(0, NCHUNK)
    def _(j):
        pltpu.make_async_copy(ones_v, t_sh.at[dst_v.at[pl.ds(0, CHUNK)]],
                              ssem).wait()

    plsc.subcore_barrier()

    pltpu.sync_copy(t_sh.at[pl.ds(lbase, NPT)], degn_v)

    @pl.loop(0, NPT // 16)
    def _(i):
        x = degn_v[pl.ds(i * 16, 16)] + 1.0
        bits = lax.bitcast_convert_type(x, jnp.int32)
        y = lax.bitcast_convert_type(0x5F3759DF - (bits >> 1), jnp.float32)
        half_x = 0.5 * x
        y = y * (1.5 - half_x * y * y)
        y = y * (1.5 - half_x * y * y)
        y = y * (1.5 - half_x * y * y)
        dinvn_v[pl.ds(i * 16, 16)] = y
        degn_v[pl.ds(i * 16, 16)] = jnp.zeros((16,), jnp.float32)

    pltpu.sync_copy(dinvn_v, dinv_sh.at[pl.ds(lbase, NPT)])
    pltpu.sync_copy(degn_v, t_sh.at[pl.ds(lbase, NPT)])

    @pl.loop(0, 8)
    def _(q):
        pltpu.sync_copy(h_hbm.at[c, pl.ds(lbase + q * (NPT // 8), NPT // 8)],
                        hbuf_v)

        @pl.loop(0, NPT // 128)
        def _(i):
            dv = dinvn_v[pl.ds(q * (NPT // 8) + i * 16, 16)]
            for k in range(16):
                r = q * (NPT // 8) + i * 16 + k
                hrow = hbuf_v[i * 16 + k, 0:F]
                hn_v[r, :] = hrow
                hrows_v[r, :] = hrow * dv[k]

    pltpu.sync_copy(hrows_v, g_hbm.at[c, pl.ds(lbase, NPT)])
    plsc.subcore_barrier()

    def gather_start(jj, buf, tbuf):
        sl = pl.ds(jj * CHUNK, CHUNK)
        pltpu.async_copy(g_hbm.at[c].at[src_v.at[sl]], buf, gsem)
        pltpu.async_copy(dinv_sh.at[dst_v.at[sl]], tbuf, tsem)

    def gather_wait(buf, tbuf):
        pltpu.make_async_copy(g_hbm.at[c].at[src_v.at[pl.ds(0, CHUNK)]], buf,
                              gsem).wait()
        pltpu.make_async_copy(dinv_sh.at[dst_v.at[pl.ds(0, CHUNK)]], tbuf,
                              tsem).wait()

    def process(jj, buf, tbuf):
        sl = pl.ds(jj * CHUNK, CHUNK)
        desc = pltpu.async_copy(buf, acc_sh.at[dst_v.at[sl]], ssem, add=True)
        pltpu.sync_copy(tbuf, t_sh.at[src_v.at[sl]], add=True)
        desc.wait()

    gather_start(0, rows0_v, tv0_v)

    @pl.loop(0, NCHUNK, step=2)
    def _(j):
        gather_wait(rows0_v, tv0_v)
        gather_start(j + 1, rows1_v, tv1_v)
        process(j, rows0_v, tv0_v)
        gather_wait(rows1_v, tv1_v)

        @pl.when(j + 2 < NCHUNK)
        def _():
            gather_start(j + 2, rows0_v, tv0_v)

        process(j + 1, rows1_v, tv1_v)

    plsc.subcore_barrier()

    pltpu.sync_copy(acc_sh.at[pl.ds(lbase, NPT)], hrows_v)
    pltpu.sync_copy(t_sh.at[pl.ds(lbase, NPT)], degn_v)
    pltpu.sync_copy(b_hbm.at[c], psum_v.at[0])
    bvec = psum_v[0, :]

    def body(i, s_acc):
        dv = dinvn_v[pl.ds(i * 16, 16)]
        inv = dv * dv
        node = lbase + i * 16 + lax.iota(jnp.int32, 16)
        cw = jnp.where(node < N, dv * degn_v[pl.ds(i * 16, 16)] + inv, 0.0)
        for k in range(16):
            h1 = jnp.maximum(
                dv[k] * hrows_v[i * 16 + k, :] + inv[k] * hn_v[i * 16 + k, :]
                + bvec, 0.0)
            s_acc = s_acc + cw[k] * h1
        return s_acc

    s_part = pl.loop(0, NPT // 16,
                     init_carry=jnp.zeros((16,), jnp.float32))(body)
    psum_v[1, :] = s_part
    pltpu.sync_copy(psum_v.at[1], part_sh.at[s])
    plsc.subcore_barrier()

    @pl.when(s == 0)
    def _():
        pltpu.sync_copy(part_sh, psum_v)
        tot = psum_v[0, :]
        for k in range(1, 16):
            tot = tot + psum_v[k, :]
        psum_v[0, :] = tot
        pltpu.sync_copy(psum_v.at[0], s_hbm.at[c])


def _mm_body(xp_ref, xl_ref, wp_ref, wl_ref, o_ref):
    zero = jnp.zeros((NP - N, F), jnp.float32)
    o_ref[0, 0:N, 0:F] = jnp.dot(xp_ref[...], wp_ref[...],
                                 preferred_element_type=jnp.float32)
    o_ref[0, N:NP, 0:F] = zero
    o_ref[1, 0:N, 0:F] = jnp.dot(xl_ref[...], wl_ref[...],
                                 preferred_element_type=jnp.float32)
    o_ref[1, N:NP, 0:F] = zero


def _head_body(s_ref, wpout_ref, bpout_ref, wlout_ref, blout_ref,
               w1_ref, b1_ref, w2_ref, b2_ref, w3_ref, b3_ref, act_ref,
               o_ref):
    p = jnp.dot(s_ref[0:1, :] / float(N), wpout_ref[...],
                preferred_element_type=jnp.float32) + bpout_ref[...]
    l = jnp.dot(s_ref[1:2, :] / float(N), wlout_ref[...],
                preferred_element_type=jnp.float32) + blout_ref[...]
    fp = jnp.maximum(
        jnp.dot(p, w1_ref[0:50, :], preferred_element_type=jnp.float32)
        + jnp.dot(l, w1_ref[50:100, :], preferred_element_type=jnp.float32)
        + b1_ref[...], 0.0)
    pol = (jnp.dot(fp, w2_ref[0:60, :], preferred_element_type=jnp.float32)
           + jnp.dot(act_ref[...], w2_ref[60:100, :],
                     preferred_element_type=jnp.float32)
           + b2_ref[...])
    o_ref[...] = jnp.dot(jnp.maximum(pol, 0.0), w3_ref[...],
                         preferred_element_type=jnp.float32) + b3_ref[...]


def kernel(protein_x, protein_edge_index, ligand_x, ligand_edge_index, action,
           W_pin, b_pin, W_pout, b_pout, W_lin, b_lin, W_lout, b_lout,
           W1, b1, W2, b2, W3, b3):
    f32 = jnp.float32
    eis = jnp.stack([
        jnp.pad(protein_edge_index, ((0, 0), (0, EP - E)), constant_values=N),
        jnp.pad(ligand_edge_index, ((0, 0), (0, EP - E)), constant_values=N),
    ])
    bs = jnp.stack([b_pin, b_lin])

    hs = pl.pallas_call(
        _mm_body,
        out_shape=jax.ShapeDtypeStruct((2, NP, 128), f32),
    )(protein_x, ligand_x, W_pin, W_lin)

    s_vec, _g = _sc_mega(eis, hs, bs)

    out = pl.pallas_call(
        _head_body,
        out_shape=jax.ShapeDtypeStruct((1, 1), f32),
    )(s_vec, W_pout, b_pout.reshape(1, 50), W_lout, b_lout.reshape(1, 50),
      W1, b1.reshape(1, 60), W2, b2.reshape(1, 10), W3, b3.reshape(1, 1),
      action)
    return out


# speedup vs baseline: 98.3002x; 1.0622x over previous
import functools

import jax
import jax.numpy as jnp
from jax import lax
from jax.experimental import pallas as pl
from jax.experimental.pallas import tpu as pltpu
from jax.experimental.pallas import tpu_sc as plsc

N = 10000
NP = 10240
E = 320000
D = 128
F = 16
NTILE = 16
CHUNK = 1024
NCHUNK = 20
EPT = NCHUNK * CHUNK
EP = EPT * NTILE
NPT = NP // NTILE
G2 = 2 * NP

_mesh = plsc.VectorSubcoreMesh(core_axis_name="core", subcore_axis_name="subcore")

_sc_params = pltpu.CompilerParams(
    needs_layout_passes=False, use_tc_tiling_on_sc=False)


@functools.partial(
    pl.kernel,
    out_type=(
        jax.ShapeDtypeStruct((2, F), jnp.float32),
        jax.ShapeDtypeStruct((2, NP, F), jnp.float32),
    ),
    mesh=_mesh,
    scratch_types=[
        pltpu.VMEM((EPT,), jnp.int32),
        pltpu.VMEM((EPT,), jnp.int32),
        pltpu.VMEM((CHUNK,), jnp.float32),
        pltpu.VMEM((CHUNK,), jnp.float32),
        pltpu.VMEM((CHUNK, F), jnp.float32),
        pltpu.VMEM((CHUNK, F), jnp.float32),
        pltpu.VMEM((NPT, F), jnp.float32),
        pltpu.VMEM((NPT, F), jnp.float32),
        pltpu.VMEM((NPT // 8, 128), jnp.float32),
        pltpu.VMEM((NPT,), jnp.float32),
        pltpu.VMEM((NPT,), jnp.float32),
        pltpu.VMEM((CHUNK,), jnp.float32),
        pltpu.VMEM((F, F), jnp.float32),
        pltpu.VMEM_SHARED((NP,), jnp.float32),
        pltpu.VMEM_SHARED((NP,), jnp.float32),
        pltpu.VMEM_SHARED((NP, F), jnp.float32),
        pltpu.VMEM_SHARED((F, F), jnp.float32),
        pltpu.SemaphoreType.DMA,
        pltpu.SemaphoreType.DMA,
        pltpu.SemaphoreType.DMA,
    ],
    compiler_params=_sc_params,
)
def _sc_mega(ei_hbm, h_hbm, b_hbm, s_hbm, g_hbm,
             src_v, dst_v, tv0_v, tv1_v, rows0_v, rows1_v, hrows_v, hn_v,
             hbuf_v, degn_v, dinvn_v, ones_v, psum_v, t_sh, dinv_sh, acc_sh,
             part_sh, gsem, ssem, tsem):
    c = lax.axis_index("core")
    s = lax.axis_index("subcore")
    base = c * NP + s * NPT
    lbase = s * NPT

    @pl.loop(0, CHUNK // 16)
    def _(i):
        ones_v[pl.ds(i * 16, 16)] = jnp.full((16,), 1.0, jnp.float32)

    @pl.loop(0, NPT)
    def _(i):
        hrows_v[i, :] = jnp.zeros((F,), jnp.float32)

    @pl.loop(0, NPT // 16)
    def _(i):
        degn_v[pl.ds(i * 16, 16)] = jnp.zeros((16,), jnp.float32)

    pltpu.sync_copy(degn_v, t_sh.at[pl.ds(lbase, NPT)])
    pltpu.sync_copy(hrows_v, acc_sh.at[pl.ds(lbase, NPT)])
    pltpu.sync_copy(ei_hbm.at[c, 0, pl.ds(s * EPT, EPT)], src_v)
    pltpu.sync_copy(ei_hbm.at[c, 1, pl.ds(s * EPT, EPT)], dst_v)
    plsc.subcore_barrier()

    @pl.loop(0, NCHUNK)
    def _(j):
        pltpu.async_copy(ones_v, t_sh.at[dst_v.at[pl.ds(j * CHUNK, CHUNK)]],
                         ssem, add=True)

    @pl.loop(0, NCHUNK)
    def _(j):
        pltpu.make_async_copy(ones_v, t_sh.at[dst_v.at[pl.ds(0, CHUNK)]],
                              ssem).wait()

    plsc.subcore_barrier()

    pltpu.sync_copy(t_sh.at[pl.ds(lbase, NPT)], degn_v)

    @pl.loop(0, NPT // 16)
    def _(i):
        x = degn_v[pl.ds(i * 16, 16)] + 1.0
        bits = lax.bitcast_convert_type(x, jnp.int32)
        y = lax.bitcast_convert_type(0x5F3759DF - (bits >> 1), jnp.float32)
        half_x = 0.5 * x
        y = y * (1.5 - half_x * y * y)
        y = y * (1.5 - half_x * y * y)
        y = y * (1.5 - half_x * y * y)
        dinvn_v[pl.ds(i * 16, 16)] = y
        degn_v[pl.ds(i * 16, 16)] = jnp.zeros((16,), jnp.float32)

    pltpu.sync_copy(dinvn_v, dinv_sh.at[pl.ds(lbase, NPT)])
    pltpu.sync_copy(degn_v, t_sh.at[pl.ds(lbase, NPT)])

    @pl.loop(0, 8)
    def _(q):
        pltpu.sync_copy(h_hbm.at[c, pl.ds(lbase + q * (NPT // 8), NPT // 8)],
                        hbuf_v)

        @pl.loop(0, NPT // 128)
        def _(i):
            dv = dinvn_v[pl.ds(q * (NPT // 8) + i * 16, 16)]
            for k in range(16):
                r = q * (NPT // 8) + i * 16 + k
                hrow = hbuf_v[i * 16 + k, 0:F]
                hn_v[r, :] = hrow
                hrows_v[r, :] = hrow * dv[k]

    pltpu.sync_copy(hrows_v, g_hbm.at[c, pl.ds(lbase, NPT)])
    plsc.subcore_barrier()

    def gather_start(jj, buf, tbuf):
        sl = pl.ds(jj * CHUNK, CHUNK)
        pltpu.async_copy(g_hbm.at[c].at[src_v.at[sl]], buf, gsem)
        pltpu.async_copy(dinv_sh.at[dst_v.at[sl]], tbuf, tsem)

    def gather_wait(buf, tbuf):
        pltpu.make_async_copy(g_hbm.at[c].at[src_v.at[pl.ds(0, CHUNK)]], buf,
                              gsem).wait()
        pltpu.make_async_copy(dinv_sh.at[dst_v.at[pl.ds(0, CHUNK)]], tbuf,
                              tsem).wait()

    def process(jj, buf, tbuf):
        sl = pl.ds(jj * CHUNK, CHUNK)
        desc = pltpu.async_copy(buf, acc_sh.at[dst_v.at[sl]], ssem, add=True)
        pltpu.sync_copy(tbuf, t_sh.at[src_v.at[sl]], add=True)
        desc.wait()

    gather_start(0, rows0_v, tv0_v)

    @pl.loop(0, NCHUNK, step=2)
    def _(j):
        gather_wait(rows0_v, tv0_v)
        gather_start(j + 1, rows1_v, tv1_v)
        process(j, rows0_v, tv0_v)
        gather_wait(rows1_v, tv1_v)

        @pl.when(j + 2 < NCHUNK)
        def _():
            gather_start(j + 2, rows0_v, tv0_v)

        process(j + 1, rows1_v, tv1_v)

    plsc.subcore_barrier()

    pltpu.sync_copy(acc_sh.at[pl.ds(lbase, NPT)], hrows_v)
    pltpu.sync_copy(t_sh.at[pl.ds(lbase, NPT)], degn_v)
    pltpu.sync_copy(b_hbm.at[c], psum_v.at[0])
    bvec = psum_v[0, :]

    def body(i, s_acc):
        dv = dinvn_v[pl.ds(i * 16, 16)]
        inv = dv * dv
        node = lbase + i * 16 + lax.iota(jnp.int32, 16)
        cw = jnp.where(node < N, dv * degn_v[pl.ds(i * 16, 16)] + inv, 0.0)
        for k in range(16):
            h1 = jnp.maximum(
                dv[k] * hrows_v[i * 16 + k, :] + inv[k] * hn_v[i * 16 + k, :]
                + bvec, 0.0)
            s_acc = s_acc + cw[k] * h1
        return s_acc

    s_part = pl.loop(0, NPT // 16,
                     init_carry=jnp.zeros((16,), jnp.float32))(body)
    psum_v[1, :] = s_part
    pltpu.sync_copy(psum_v.at[1], part_sh.at[s])
    plsc.subcore_barrier()

    @pl.when(s == 0)
    def _():
        pltpu.sync_copy(part_sh, psum_v)
        tot = psum_v[0, :]
        for k in range(1, 16):
            tot = tot + psum_v[k, :]
        psum_v[0, :] = tot
        pltpu.sync_copy(psum_v.at[0], s_hbm.at[c])


def _mm_body(xp_ref, xl_ref, wp_ref, wl_ref, o_ref):
    zero = jnp.zeros((NP - N, F), jnp.float32)
    o_ref[0, 0:N, 0:F] = jnp.dot(xp_ref[...], wp_ref[...],
                                 preferred_element_type=jnp.float32)
    o_ref[0, N:NP, 0:F] = zero
    o_ref[1, 0:N, 0:F] = jnp.dot(xl_ref[...], wl_ref[...],
                                 preferred_element_type=jnp.float32)
    o_ref[1, N:NP, 0:F] = zero


def _head_body(s_ref, wpout_ref, bpout_ref, wlout_ref, blout_ref,
               w1_ref, b1_ref, w2_ref, b2_ref, w3_ref, b3_ref, act_ref,
               o_ref):
    p = jnp.dot(s_ref[0:1, :] / float(N), wpout_ref[...],
                preferred_element_type=jnp.float32) + bpout_ref[...]
    l = jnp.dot(s_ref[1:2, :] / float(N), wlout_ref[...],
                preferred_element_type=jnp.float32) + blout_ref[...]
    fp = jnp.maximum(
        jnp.dot(p, w1_ref[0:50, :], preferred_element_type=jnp.float32)
        + jnp.dot(l, w1_ref[50:100, :], preferred_element_type=jnp.float32)
        + b1_ref[...], 0.0)
    pol = (jnp.dot(fp, w2_ref[0:60, :], preferred_element_type=jnp.float32)
           + jnp.dot(act_ref[...], w2_ref[60:100, :],
                     preferred_element_type=jnp.float32)
           + b2_ref[...])
    o_ref[...] = jnp.dot(jnp.maximum(pol, 0.0), w3_ref[...],
                         preferred_element_type=jnp.float32) + b3_ref[...]


def kernel(protein_x, protein_edge_index, ligand_x, ligand_edge_index, action,
           W_pin, b_pin, W_pout, b_pout, W_lin, b_lin, W_lout, b_lout,
           W1, b1, W2, b2, W3, b3):
    f32 = jnp.float32
    eis = jnp.stack([
        jnp.pad(protein_edge_index, ((0, 0), (0, EP - E)), constant_values=N),
        jnp.pad(ligand_edge_index, ((0, 0), (0, EP - E)), constant_values=N),
    ])
    bs = jnp.stack([b_pin, b_lin])

    hs = pl.pallas_call(
        _mm_body,
        out_shape=jax.ShapeDtypeStruct((2, NP, 128), f32),
    )(protein_x, ligand_x, W_pin, W_lin)

    s_vec, _g = _sc_mega(eis, hs, bs)

    out = pl.pallas_call(
        _head_body,
        out_shape=jax.ShapeDtypeStruct((1, 1), f32),
    )(s_vec, W_pout, b_pout.reshape(1, 50), W_lout, b_lout.reshape(1, 50),
      W1, b1.reshape(1, 60), W2, b2.reshape(1, 10), W3, b3.reshape(1, 1),
      action)
    return out


# speedup vs baseline: 111.8553x; 1.1379x over previous
import functools

import jax
import jax.numpy as jnp
from jax import lax
from jax.experimental import pallas as pl
from jax.experimental.pallas import tpu as pltpu
from jax.experimental.pallas import tpu_sc as plsc

N = 10000
NP = 10240
E = 320000
D = 128
F = 16
NTILE = 16
CHUNK = 1024
NCHUNK = 20
EPT = NCHUNK * CHUNK
EP = EPT * NTILE
NPT = NP // NTILE
G2 = 2 * NP

_mesh = plsc.VectorSubcoreMesh(core_axis_name="core", subcore_axis_name="subcore")

_sc_params = pltpu.CompilerParams(
    needs_layout_passes=False, use_tc_tiling_on_sc=False)


@functools.partial(
    pl.kernel,
    out_type=jax.ShapeDtypeStruct((2, F), jnp.float32),
    mesh=_mesh,
    scratch_types=[
        pltpu.VMEM((EPT,), jnp.int32),
        pltpu.VMEM((EPT,), jnp.int32),
        pltpu.VMEM((CHUNK,), jnp.float32),
        pltpu.VMEM((CHUNK,), jnp.float32),
        pltpu.VMEM((CHUNK, F), jnp.float32),
        pltpu.VMEM((CHUNK, F), jnp.float32),
        pltpu.VMEM((NPT, F), jnp.float32),
        pltpu.VMEM((NPT, F), jnp.float32),
        pltpu.VMEM((NPT // 8, 128), jnp.float32),
        pltpu.VMEM((NPT,), jnp.float32),
        pltpu.VMEM((NPT,), jnp.float32),
        pltpu.VMEM((CHUNK,), jnp.float32),
        pltpu.VMEM((F, F), jnp.float32),
        pltpu.VMEM_SHARED((NP,), jnp.float32),
        pltpu.VMEM_SHARED((NP,), jnp.float32),
        pltpu.VMEM_SHARED((NP, F), jnp.float32),
        pltpu.VMEM_SHARED((NP, F), jnp.float32),
        pltpu.VMEM_SHARED((F, F), jnp.float32),
        pltpu.SemaphoreType.DMA,
        pltpu.SemaphoreType.DMA,
        pltpu.SemaphoreType.DMA,
    ],
    compiler_params=_sc_params,
)
def _sc_mega(ei_hbm, h_hbm, b_hbm, s_hbm,
             src_v, dst_v, tv0_v, tv1_v, rows0_v, rows1_v, hrows_v, hn_v,
             hbuf_v, degn_v, dinvn_v, ones_v, psum_v, t_sh, dinv_sh, acc_sh,
             g_sh, part_sh, gsem, ssem, tsem):
    c = lax.axis_index("core")
    s = lax.axis_index("subcore")
    base = c * NP + s * NPT
    lbase = s * NPT

    @pl.loop(0, CHUNK // 16)
    def _(i):
        ones_v[pl.ds(i * 16, 16)] = jnp.full((16,), 1.0, jnp.float32)

    @pl.loop(0, NPT)
    def _(i):
        hrows_v[i, :] = jnp.zeros((F,), jnp.float32)

    @pl.loop(0, NPT // 16)
    def _(i):
        degn_v[pl.ds(i * 16, 16)] = jnp.zeros((16,), jnp.float32)

    pltpu.sync_copy(degn_v, t_sh.at[pl.ds(lbase, NPT)])
    pltpu.sync_copy(hrows_v, acc_sh.at[pl.ds(lbase, NPT)])
    pltpu.sync_copy(ei_hbm.at[c, 0, pl.ds(s * EPT, EPT)], src_v)
    pltpu.sync_copy(ei_hbm.at[c, 1, pl.ds(s * EPT, EPT)], dst_v)
    plsc.subcore_barrier()

    @pl.loop(0, NCHUNK)
    def _(j):
        pltpu.async_copy(ones_v, t_sh.at[dst_v.at[pl.ds(j * CHUNK, CHUNK)]],
                         ssem, add=True)

    @pl.loop(0, NCHUNK)
    def _(j):
        pltpu.make_async_copy(ones_v, t_sh.at[dst_v.at[pl.ds(0, CHUNK)]],
                              ssem).wait()

    plsc.subcore_barrier()

    pltpu.sync_copy(t_sh.at[pl.ds(lbase, NPT)], degn_v)

    @pl.loop(0, NPT // 16)
    def _(i):
        x = degn_v[pl.ds(i * 16, 16)] + 1.0
        bits = lax.bitcast_convert_type(x, jnp.int32)
        y = lax.bitcast_convert_type(0x5F3759DF - (bits >> 1), jnp.float32)
        half_x = 0.5 * x
        y = y * (1.5 - half_x * y * y)
        y = y * (1.5 - half_x * y * y)
        y = y * (1.5 - half_x * y * y)
        dinvn_v[pl.ds(i * 16, 16)] = y
        degn_v[pl.ds(i * 16, 16)] = jnp.zeros((16,), jnp.float32)

    pltpu.sync_copy(dinvn_v, dinv_sh.at[pl.ds(lbase, NPT)])
    pltpu.sync_copy(degn_v, t_sh.at[pl.ds(lbase, NPT)])

    @pl.loop(0, 8)
    def _(q):
        pltpu.sync_copy(h_hbm.at[c, pl.ds(lbase + q * (NPT // 8), NPT // 8)],
                        hbuf_v)

        @pl.loop(0, NPT // 128)
        def _(i):
            dv = dinvn_v[pl.ds(q * (NPT // 8) + i * 16, 16)]
            for k in range(16):
                r = q * (NPT // 8) + i * 16 + k
                hrow = hbuf_v[i * 16 + k, 0:F]
                hn_v[r, :] = hrow
                hrows_v[r, :] = hrow * dv[k]

    pltpu.sync_copy(hrows_v, g_sh.at[pl.ds(lbase, NPT)])
    plsc.subcore_barrier()

    def gather_start(jj, buf, tbuf):
        sl = pl.ds(jj * CHUNK, CHUNK)
        pltpu.async_copy(g_sh.at[src_v.at[sl]], buf, gsem)
        pltpu.async_copy(dinv_sh.at[dst_v.at[sl]], tbuf, tsem)

    def gather_wait(buf, tbuf):
        pltpu.make_async_copy(g_sh.at[src_v.at[pl.ds(0, CHUNK)]], buf,
                              gsem).wait()
        pltpu.make_async_copy(dinv_sh.at[dst_v.at[pl.ds(0, CHUNK)]], tbuf,
                              tsem).wait()

    def process(jj, buf, tbuf):
        sl = pl.ds(jj * CHUNK, CHUNK)
        desc = pltpu.async_copy(buf, acc_sh.at[dst_v.at[sl]], ssem, add=True)
        pltpu.sync_copy(tbuf, t_sh.at[src_v.at[sl]], add=True)
        desc.wait()

    gather_start(0, rows0_v, tv0_v)

    @pl.loop(0, NCHUNK, step=2)
    def _(j):
        gather_wait(rows0_v, tv0_v)
        gather_start(j + 1, rows1_v, tv1_v)
        process(j, rows0_v, tv0_v)
        gather_wait(rows1_v, tv1_v)

        @pl.when(j + 2 < NCHUNK)
        def _():
            gather_start(j + 2, rows0_v, tv0_v)

        process(j + 1, rows1_v, tv1_v)

    plsc.subcore_barrier()

    pltpu.sync_copy(acc_sh.at[pl.ds(lbase, NPT)], hrows_v)
    pltpu.sync_copy(t_sh.at[pl.ds(lbase, NPT)], degn_v)
    pltpu.sync_copy(b_hbm.at[c], psum_v.at[0])
    bvec = psum_v[0, :]

    def body(i, s_acc):
        dv = dinvn_v[pl.ds(i * 16, 16)]
        inv = dv * dv
        node = lbase + i * 16 + lax.iota(jnp.int32, 16)
        cw = jnp.where(node < N, dv * degn_v[pl.ds(i * 16, 16)] + inv, 0.0)
        for k in range(16):
            h1 = jnp.maximum(
                dv[k] * hrows_v[i * 16 + k, :] + inv[k] * hn_v[i * 16 + k, :]
                + bvec, 0.0)
            s_acc = s_acc + cw[k] * h1
        return s_acc

    s_part = pl.loop(0, NPT // 16,
                     init_carry=jnp.zeros((16,), jnp.float32))(body)
    psum_v[1, :] = s_part
    pltpu.sync_copy(psum_v.at[1], part_sh.at[s])
    plsc.subcore_barrier()

    @pl.when(s == 0)
    def _():
        pltpu.sync_copy(part_sh, psum_v)
        tot = psum_v[0, :]
        for k in range(1, 16):
            tot = tot + psum_v[k, :]
        psum_v[0, :] = tot
        pltpu.sync_copy(psum_v.at[0], s_hbm.at[c])


def _mm_body(xp_ref, xl_ref, wp_ref, wl_ref, o_ref):
    zero = jnp.zeros((NP - N, F), jnp.float32)
    o_ref[0, 0:N, 0:F] = jnp.dot(xp_ref[...], wp_ref[...],
                                 preferred_element_type=jnp.float32)
    o_ref[0, N:NP, 0:F] = zero
    o_ref[1, 0:N, 0:F] = jnp.dot(xl_ref[...], wl_ref[...],
                                 preferred_element_type=jnp.float32)
    o_ref[1, N:NP, 0:F] = zero


def _head_body(s_ref, wpout_ref, bpout_ref, wlout_ref, blout_ref,
               w1_ref, b1_ref, w2_ref, b2_ref, w3_ref, b3_ref, act_ref,
               o_ref):
    p = jnp.dot(s_ref[0:1, :] / float(N), wpout_ref[...],
                preferred_element_type=jnp.float32) + bpout_ref[...]
    l = jnp.dot(s_ref[1:2, :] / float(N), wlout_ref[...],
                preferred_element_type=jnp.float32) + blout_ref[...]
    fp = jnp.maximum(
        jnp.dot(p, w1_ref[0:50, :], preferred_element_type=jnp.float32)
        + jnp.dot(l, w1_ref[50:100, :], preferred_element_type=jnp.float32)
        + b1_ref[...], 0.0)
    pol = (jnp.dot(fp, w2_ref[0:60, :], preferred_element_type=jnp.float32)
           + jnp.dot(act_ref[...], w2_ref[60:100, :],
                     preferred_element_type=jnp.float32)
           + b2_ref[...])
    o_ref[...] = jnp.dot(jnp.maximum(pol, 0.0), w3_ref[...],
                         preferred_element_type=jnp.float32) + b3_ref[...]


def kernel(protein_x, protein_edge_index, ligand_x, ligand_edge_index, action,
           W_pin, b_pin, W_pout, b_pout, W_lin, b_lin, W_lout, b_lout,
           W1, b1, W2, b2, W3, b3):
    f32 = jnp.float32
    eis = jnp.stack([
        jnp.pad(protein_edge_index, ((0, 0), (0, EP - E)), constant_values=N),
        jnp.pad(ligand_edge_index, ((0, 0), (0, EP - E)), constant_values=N),
    ])
    bs = jnp.stack([b_pin, b_lin])

    hs = pl.pallas_call(
        _mm_body,
        out_shape=jax.ShapeDtypeStruct((2, NP, 128), f32),
    )(protein_x, ligand_x, W_pin, W_lin)

    s_vec = _sc_mega(eis, hs, bs)

    out = pl.pallas_call(
        _head_body,
        out_shape=jax.ShapeDtypeStruct((1, 1), f32),
    )(s_vec, W_pout, b_pout.reshape(1, 50), W_lout, b_lout.reshape(1, 50),
      W1, b1.reshape(1, 60), W2, b2.reshape(1, 10), W3, b3.reshape(1, 1),
      action)
    return out
